# Initial kernel scaffold; baseline (speedup 1.0000x reference)
import jax
import jax.numpy as jnp
from jax.experimental import pallas as pl


def kernel(x, edge_index, batch, W0, b0, g0, be0, W1, as1, ad1, b1, g1, be1, W2, as2, ad2, b2, g2, be2, Wna, bna, Wea, bea, Wc, bc, Wt, bt, Wc1, bc1, gc, bec, Wc2, bc2, Wt1, bt1, gt, bet, Wt2, bt2, Wo1, bo1, go, beo, Wo2, bo2):
    raise NotImplementedError("write your pallas kernel here")


# speedup vs baseline: 1.0496x; 1.0496x over previous
import functools

import jax
import jax.numpy as jnp
from jax import lax
from jax.experimental import pallas as pl
from jax.experimental.pallas import tpu as pltpu

N = 10000
E = 160000
F_IN = 128
HID = 64
HEADS = 8
D = HID * HEADS
G = 64
NC = 10

NPAD = 10240


def _mm_body(x_ref, w_ref, b_ref, o_ref):
    o_ref[...] = (
        jnp.dot(x_ref[...], w_ref[...], preferred_element_type=jnp.float32)
        + b_ref[...]
    )


def _matmul_bias(x, w, b, bm=512):
    m, k = x.shape
    k2, n = w.shape
    assert k == k2
    grid = (m // bm,)
    return pl.pallas_call(
        _mm_body,
        grid=grid,
        in_specs=[
            pl.BlockSpec((bm, k), lambda i: (i, 0)),
            pl.BlockSpec((k, n), lambda i: (0, 0)),
            pl.BlockSpec((1, n), lambda i: (0, 0)),
        ],
        out_specs=pl.BlockSpec((bm, n), lambda i: (i, 0)),
        out_shape=jax.ShapeDtypeStruct((m, n), jnp.float32),
    )(x, w, b.reshape(1, n))


def _pad_rows(x, rows):
    return jnp.pad(x, ((0, rows - x.shape[0]), (0, 0)))


def _bn(x, g, b):
    m = x.mean(axis=0)
    v = x.var(axis=0)
    return (x - m) / jnp.sqrt(v + 1e-5) * g + b


def _bn_n(xp):
    s = xp.sum(axis=0)
    m = s / N
    v = (xp * xp).sum(axis=0) / N - m * m
    return m, v


def _gcn_edges(h, src, dst, ew):
    n = h.shape[0]
    loop = jnp.arange(n)
    s = jnp.concatenate([src, loop])
    d = jnp.concatenate([dst, loop])
    w = jnp.concatenate([ew, jnp.ones((n,), h.dtype)])
    deg = jax.ops.segment_sum(w, d, num_segments=n)
    dis = jnp.where(deg > 0, 1.0 / jnp.sqrt(jnp.maximum(deg, 1e-12)), 0.0)
    norm = dis[s] * w * dis[d]
    return jax.ops.segment_sum(h[s] * norm[:, None], d, num_segments=n)


def _gat_edges(h3, es, ed, src, dst):
    n = h3.shape[0]
    loop = jnp.arange(n)
    s = jnp.concatenate([src, loop])
    d = jnp.concatenate([dst, loop])
    a = jax.nn.leaky_relu(es[s] + ed[d], 0.2)
    amax = es.max(axis=0) + ed.max(axis=0)
    p = jnp.exp(a - amax[None, :])
    den = jax.ops.segment_sum(p, d, num_segments=n)
    acc = jax.ops.segment_sum(h3[s] * p[:, :, None], d, num_segments=n)
    return acc / den[:, :, None]


def _pool(x, batch):
    s = jax.ops.segment_sum(x, batch, num_segments=G)
    c = jax.ops.segment_sum(jnp.ones((x.shape[0],), x.dtype), batch,
                            num_segments=G)
    return s / jnp.maximum(c, 1.0)[:, None]


def _clf(x, W1, b1, g, be, W2, b2):
    h = jax.nn.relu(_bn(x @ W1 + b1, g, be))
    return h @ W2 + b2


def kernel(x, edge_index, batch, W0, b0, g0, be0, W1, as1, ad1, b1, g1, be1,
           W2, as2, ad2, b2, g2, be2, Wna, bna, Wea, bea, Wc, bc, Wt, bt,
           Wc1, bc1, gc, bec, Wc2, bc2, Wt1, bt1, gt, bet, Wt2, bt2,
           Wo1, bo1, go, beo, Wo2, bo2):
    src = edge_index[0]
    dst = edge_index[1]

    xp = _pad_rows(x, NPAD)

    h0 = _matmul_bias(xp, W0, jnp.zeros((HID,), jnp.float32))[:N]
    a0 = _gcn_edges(h0, src, dst, jnp.ones((E,), jnp.float32)) + b0
    h = jax.nn.relu(a0)
    h = _bn(h, g0, be0)

    hw = _matmul_bias(_pad_rows(h, NPAD), W1, jnp.zeros((D,), jnp.float32))[:N]
    h3 = hw.reshape(N, HEADS, HID)
    es = (h3 * as1[None]).sum(-1)
    ed = (h3 * ad1[None]).sum(-1)
    g_out = _gat_edges(h3, es, ed, src, dst).reshape(N, D) + b1
    h = jax.nn.elu(_bn(g_out, g1, be1))

    hw = _matmul_bias(_pad_rows(h, NPAD), W2, jnp.zeros((D,), jnp.float32))[:N]
    h3 = hw.reshape(N, HEADS, HID)
    es = (h3 * as2[None]).sum(-1)
    ed = (h3 * ad2[None]).sum(-1)
    g_out = _gat_edges(h3, es, ed, src, dst).reshape(N, D) + b2
    h = jax.nn.elu(_bn(g_out, g2, be2))

    u = h @ Wea[:D]
    v = h @ Wea[D:]
    logits = u[src] + v[dst] + bea
    ea = jax.nn.softmax(logits, axis=-1)

    hc = _matmul_bias(_pad_rows(h, NPAD), Wc, jnp.zeros((D,), jnp.float32))[:N]
    ht = _matmul_bias(_pad_rows(h, NPAD), Wt, jnp.zeros((D,), jnp.float32))[:N]
    causal = _gcn_edges(hc, src, dst, ea[:, 0]) + bc
    trivial = _gcn_edges(ht, src, dst, ea[:, 1]) + bt

    pc = _pool(causal, batch)
    pt = _pool(trivial, batch)
    comb = jnp.concatenate([pc, pt], axis=1)
    oc = jax.nn.log_softmax(_clf(pc, Wc1, bc1, gc, bec, Wc2, bc2), axis=-1)
    ot = jax.nn.log_softmax(_clf(pt, Wt1, bt1, gt, bet, Wt2, bt2), axis=-1)
    oco = jax.nn.log_softmax(_clf(comb, Wo1, bo1, go, beo, Wo2, bo2), axis=-1)
    return (oc, ot, oco)


# speedup vs baseline: 8.9561x; 8.5330x over previous
import functools

import jax
import jax.numpy as jnp
from jax import lax
from jax.experimental import pallas as pl
from jax.experimental.pallas import tpu as pltpu
from jax.experimental.pallas import tpu_sc as plsc

N = 10000
E = 160000
F_IN = 128
HID = 64
HEADS = 8
D = HID * HEADS
G = 64
NC = 10
EPS = 1e-5

BM = 1000
NW = 32
E_PAD = E + 256

_mesh = plsc.VectorSubcoreMesh(core_axis_name="c", subcore_axis_name="s")


def _dyng(v, idx):
    return lax.gather(
        v, idx[:, None],
        lax.GatherDimensionNumbers(offset_dims=(), collapsed_slice_dims=(0,),
                                   start_index_map=(0,)),
        (1,), mode=lax.GatherScatterMode.PROMISE_IN_BOUNDS)


def _splat(v, lane):
    return _dyng(v, jnp.full((16,), lane, jnp.int32))


def _zero_rows(z_v, rows, cols):
    def zr(i, _):
        for k in range(cols // 16):
            z_v[i, pl.ds(k * 16, 16)] = jnp.zeros((16,), jnp.float32)
        return 0
    lax.fori_loop(0, rows, zr, 0)


def _make_sc_deg(n, e):
    epw = e // NW
    npad = -(-n // 1280) * 1280
    rpt = npad // 16
    zr = 32
    nb, tail = divmod(epw, 16)

    @functools.partial(
        pl.kernel,
        out_type=jax.ShapeDtypeStruct((2, npad, 128), jnp.float32),
        mesh=_mesh,
        scratch_types=[pltpu.VMEM((epw,), jnp.int32),
                       pltpu.VMEM((16, 128), jnp.float32),
                       pltpu.VMEM((zr, 128), jnp.float32),
                       pltpu.VMEM_SHARED((npad, 128), jnp.float32),
                       pltpu.SemaphoreType.DMA])
    def k(dst_hbm, out_hbm, dst_v, one_v, z_v, acc_sh, sem):
        cid = lax.axis_index("c")
        sid = lax.axis_index("s")
        wid = cid * 16 + sid
        _zero_rows(z_v, zr, 128)
        for j in range(rpt // zr):
            pltpu.sync_copy(z_v, acc_sh.at[pl.ds(sid * rpt + j * zr, zr)])
        iota = lax.iota(jnp.int32, 16)
        ones0 = jnp.where(iota == 0, 1.0, 0.0)
        for i in range(16):
            one_v[i, pl.ds(0, 16)] = ones0
            for kk in range(1, 8):
                one_v[i, pl.ds(kk * 16, 16)] = jnp.zeros((16,), jnp.float32)
        plsc.subcore_barrier()
        pltpu.sync_copy(dst_hbm.at[pl.ds(wid * epw, epw)], dst_v)

        def blk(j, _):
            didx = dst_v[pl.ds(j * 16, 16)]
            pltpu.sync_copy(one_v, acc_sh.at[didx], add=True)
            return 0
        lax.fori_loop(0, nb, blk, 0)
        if tail:
            didx = dst_v[pl.ds(nb * 16, tail)]
            pltpu.sync_copy(one_v.at[pl.ds(0, tail)], acc_sh.at[didx], add=True)
        plsc.subcore_barrier()
        for j in range(rpt // zr):
            r0 = sid * rpt + j * zr
            pltpu.sync_copy(acc_sh.at[pl.ds(r0, zr)], out_hbm.at[cid, pl.ds(r0, zr)])
    return k


def _make_sc_gcn0(n, e):
    epw = e // NW
    npad = -(-n // 1280) * 1280
    rpt = npad // 16
    zr = 32
    nb, tail = divmod(epw, 16)

    @functools.partial(
        pl.kernel,
        out_type=jax.ShapeDtypeStruct((2, npad, 128), jnp.float32),
        mesh=_mesh,
        scratch_types=[pltpu.VMEM((epw,), jnp.int32),
                       pltpu.VMEM((epw,), jnp.int32),
                       pltpu.VMEM((16, 128), jnp.float32),
                       pltpu.VMEM((zr, 128), jnp.float32),
                       pltpu.VMEM_SHARED((npad, 128), jnp.float32),
                       pltpu.SemaphoreType.DMA])
    def k(tab_hbm, src_hbm, dst_hbm, out_hbm, src_v, dst_v, rows_v, z_v,
          acc_sh, sem):
        cid = lax.axis_index("c")
        sid = lax.axis_index("s")
        wid = cid * 16 + sid
        _zero_rows(z_v, zr, 128)
        for j in range(rpt // zr):
            pltpu.sync_copy(z_v, acc_sh.at[pl.ds(sid * rpt + j * zr, zr)])
        plsc.subcore_barrier()
        pltpu.sync_copy(src_hbm.at[pl.ds(wid * epw, epw)], src_v)
        pltpu.sync_copy(dst_hbm.at[pl.ds(wid * epw, epw)], dst_v)

        def blk(j, _):
            sidx = src_v[pl.ds(j * 16, 16)]
            pltpu.async_copy(tab_hbm.at[sidx], rows_v, sem).wait()
            didx = dst_v[pl.ds(j * 16, 16)]
            pltpu.sync_copy(rows_v, acc_sh.at[didx], add=True)
            return 0
        lax.fori_loop(0, nb, blk, 0)
        if tail:
            sidx = src_v[pl.ds(nb * 16, tail)]
            pltpu.async_copy(tab_hbm.at[sidx], rows_v.at[pl.ds(0, tail)], sem).wait()
            didx = dst_v[pl.ds(nb * 16, tail)]
            pltpu.sync_copy(rows_v.at[pl.ds(0, tail)], acc_sh.at[didx], add=True)
        plsc.subcore_barrier()
        for j in range(rpt // zr):
            r0 = sid * rpt + j * zr
            pltpu.sync_copy(acc_sh.at[pl.ds(r0, zr)], out_hbm.at[cid, pl.ds(r0, zr)])
    return k


def _make_sc_att(n, e):
    epw = e // NW
    npad = -(-n // 1280) * 1280
    rpt = npad // 16
    zr = 32
    nb, tail = divmod(epw, 16)

    @functools.partial(
        pl.kernel,
        out_type=(jax.ShapeDtypeStruct((e * 16,), jnp.float32),
                  jax.ShapeDtypeStruct((2, npad, 128), jnp.float32)),
        mesh=_mesh,
        scratch_types=[pltpu.VMEM((epw,), jnp.int32),
                       pltpu.VMEM((epw,), jnp.int32),
                       pltpu.VMEM((16, 128), jnp.float32),
                       pltpu.VMEM((16, 128), jnp.float32),
                       pltpu.VMEM((256,), jnp.float32),
                       pltpu.VMEM((16, 128), jnp.float32),
                       pltpu.VMEM((zr, 128), jnp.float32),
                       pltpu.VMEM_SHARED((npad, 128), jnp.float32),
                       pltpu.SemaphoreType.DMA,
                       pltpu.SemaphoreType.DMA])
    def k(t_hbm, src_hbm, dst_hbm, p_hbm, den_hbm,
          src_v, dst_v, rs_v, rd_v, pb_v, db_v, z_v, acc_sh, sem, sem2):
        cid = lax.axis_index("c")
        sid = lax.axis_index("s")
        wid = cid * 16 + sid
        base = wid * epw
        _zero_rows(z_v, zr, 128)
        for j in range(rpt // zr):
            pltpu.sync_copy(z_v, acc_sh.at[pl.ds(sid * rpt + j * zr, zr)])
        _zero_rows(db_v, 16, 128)
        plsc.subcore_barrier()
        pltpu.sync_copy(src_hbm.at[pl.ds(base, epw)], src_v)
        pltpu.sync_copy(dst_hbm.at[pl.ds(base, epw)], dst_v)
        iota = lax.iota(jnp.int32, 16)
        rot = iota % 8 + 8

        def do_blk(j, nrow):
            sidx = src_v[pl.ds(j * 16, nrow)]
            didx = dst_v[pl.ds(j * 16, nrow)]
            c1 = pltpu.async_copy(t_hbm.at[sidx], rs_v.at[pl.ds(0, nrow)], sem)
            c2 = pltpu.async_copy(t_hbm.at[didx], rd_v.at[pl.ds(0, nrow)], sem2)
            c1.wait()
            c2.wait()
            for ee in range(nrow):
                es = rs_v[ee, pl.ds(0, 16)]
                ed = _dyng(rd_v[ee, pl.ds(0, 16)], rot)
                a = es + ed
                a = jnp.maximum(a, 0.2 * a)
                p = jnp.where(iota < 8, jnp.exp(a), 0.0)
                pb_v[pl.ds(ee * 16, 16)] = p
                db_v[ee, pl.ds(0, 16)] = p
            pltpu.sync_copy(pb_v,
                            p_hbm.at[pl.ds((base + j * 16) * 16, 256)])
            pltpu.sync_copy(db_v.at[pl.ds(0, nrow)], acc_sh.at[didx], add=True)

        def blk(j, _):
            do_blk(j, 16)
            return 0
        lax.fori_loop(0, nb, blk, 0)
        if tail:
            do_blk(nb, tail)
        plsc.subcore_barrier()
        for j in range(rpt // zr):
            r0 = sid * rpt + j * zr
            pltpu.sync_copy(acc_sh.at[pl.ds(r0, zr)], den_hbm.at[cid, pl.ds(r0, zr)])
    return k


def _make_sc_ea(n, e):
    epw = e // NW
    npad = -(-n // 1280) * 1280
    rpt = npad // 16
    zr = 32
    nb, tail = divmod(epw, 16)

    @functools.partial(
        pl.kernel,
        out_type=(jax.ShapeDtypeStruct((e * 16,), jnp.float32),
                  jax.ShapeDtypeStruct((2, npad, 128), jnp.float32)),
        mesh=_mesh,
        scratch_types=[pltpu.VMEM((epw,), jnp.int32),
                       pltpu.VMEM((epw,), jnp.int32),
                       pltpu.VMEM((16, 128), jnp.float32),
                       pltpu.VMEM((16, 128), jnp.float32),
                       pltpu.VMEM((256,), jnp.float32),
                       pltpu.VMEM((16, 128), jnp.float32),
                       pltpu.VMEM((zr, 128), jnp.float32),
                       pltpu.VMEM_SHARED((npad, 128), jnp.float32),
                       pltpu.SemaphoreType.DMA,
                       pltpu.SemaphoreType.DMA])
    def k(t_hbm, src_hbm, dst_hbm, ea_hbm, deg_hbm,
          src_v, dst_v, rs_v, rd_v, eb_v, db_v, z_v, acc_sh, sem, sem2):
        cid = lax.axis_index("c")
        sid = lax.axis_index("s")
        wid = cid * 16 + sid
        base = wid * epw
        _zero_rows(z_v, zr, 128)
        for j in range(rpt // zr):
            pltpu.sync_copy(z_v, acc_sh.at[pl.ds(sid * rpt + j * zr, zr)])
        _zero_rows(db_v, 16, 128)
        plsc.subcore_barrier()
        pltpu.sync_copy(src_hbm.at[pl.ds(base, epw)], src_v)
        pltpu.sync_copy(dst_hbm.at[pl.ds(base, epw)], dst_v)
        iota = lax.iota(jnp.int32, 16)
        rot2 = iota % 2 + 2

        def do_blk(j, nrow):
            sidx = src_v[pl.ds(j * 16, nrow)]
            didx = dst_v[pl.ds(j * 16, nrow)]
            c1 = pltpu.async_copy(t_hbm.at[sidx], rs_v.at[pl.ds(0, nrow)], sem)
            c2 = pltpu.async_copy(t_hbm.at[didx], rd_v.at[pl.ds(0, nrow)], sem2)
            c1.wait()
            c2.wait()
            for ee in range(nrow):
                l = rs_v[ee, pl.ds(0, 16)] + _dyng(rd_v[ee, pl.ds(0, 16)], rot2)
                ldiff = _splat(l, 1) - _splat(l, 0)
                ea0 = 1.0 / (1.0 + jnp.exp(ldiff))
                ea1 = 1.0 - ea0
                eb_v[pl.ds(ee * 16, 16)] = jnp.where(iota < 8, ea0, ea1)
                db_v[ee, pl.ds(0, 16)] = jnp.where(
                    iota == 0, ea0, jnp.where(iota == 1, ea1, 0.0))
            pltpu.sync_copy(eb_v,
                            ea_hbm.at[pl.ds((base + j * 16) * 16, 256)])
            pltpu.sync_copy(db_v.at[pl.ds(0, nrow)], acc_sh.at[didx], add=True)

        def blk(j, _):
            do_blk(j, 16)
            return 0
        lax.fori_loop(0, nb, blk, 0)
        if tail:
            do_blk(nb, tail)
        plsc.subcore_barrier()
        for j in range(rpt // zr):
            r0 = sid * rpt + j * zr
            pltpu.sync_copy(acc_sh.at[pl.ds(r0, zr)], deg_hbm.at[cid, pl.ds(r0, zr)])
    return k


def _make_sc_passb(n, e, nq, qbase):
    ept = e // 16
    grp = 2000
    ngrp = ept // grp
    nbg = grp // 16
    npad = -(-n // 1280) * 1280
    rpt = npad // 16
    zr = 32

    @functools.partial(
        pl.kernel,
        out_type=jax.ShapeDtypeStruct((npad, 512), jnp.float32),
        mesh=_mesh,
        scratch_types=[pltpu.VMEM((grp,), jnp.int32),
                       pltpu.VMEM((grp,), jnp.int32),
                       pltpu.VMEM((256,), jnp.float32),
                       pltpu.VMEM((16, 128), jnp.float32),
                       pltpu.VMEM((16, 128), jnp.float32),
                       pltpu.VMEM((zr, 128), jnp.float32),
                       pltpu.VMEM_SHARED((npad, 128), jnp.float32),
                       pltpu.SemaphoreType.DMA])
    def k(h_hbm, src_hbm, dst_hbm, p_hbm, out_hbm,
          src_v, dst_v, p_v, rows_v, ob_v, z_v, acc_sh, sem):
        cid = lax.axis_index("c")
        sid = lax.axis_index("s")
        ebase = sid * ept
        for kq in range(2):
            qt = cid * 2 + kq
            q = qbase + qt
            lane0 = jnp.full((16,), 2 * q, jnp.int32)
            lane1 = lane0 + 1
            _zero_rows(z_v, zr, 128)
            for j in range(rpt // zr):
                pltpu.sync_copy(z_v, acc_sh.at[pl.ds(sid * rpt + j * zr, zr)])
            plsc.subcore_barrier()

            def grp_body(g, _):
                goff = ebase + g * grp
                pltpu.sync_copy(src_hbm.at[pl.ds(goff, grp)], src_v)
                pltpu.sync_copy(dst_hbm.at[pl.ds(goff, grp)], dst_v)

                def blk(j, _):
                    sidx = src_v[pl.ds(j * 16, 16)]
                    cg = pltpu.async_copy(h_hbm.at[q].at[sidx], rows_v, sem)
                    pltpu.sync_copy(p_hbm.at[pl.ds((goff + j * 16) * 16, 256)],
                                    p_v)
                    cg.wait()
                    for ee in range(16):
                        pr = p_v[pl.ds(ee * 16, 16)]
                        w0 = _dyng(pr, lane0)
                        w1 = _dyng(pr, lane1)
                        for v in range(4):
                            ob_v[ee, pl.ds(v * 16, 16)] = (
                                rows_v[ee, pl.ds(v * 16, 16)] * w0)
                        for v in range(4, 8):
                            ob_v[ee, pl.ds(v * 16, 16)] = (
                                rows_v[ee, pl.ds(v * 16, 16)] * w1)
                    didx = dst_v[pl.ds(j * 16, 16)]
                    pltpu.sync_copy(ob_v, acc_sh.at[didx], add=True)
                    return 0
                lax.fori_loop(0, nbg, blk, 0)
                return 0
            lax.fori_loop(0, ngrp, grp_body, 0)
            plsc.subcore_barrier()
            for j in range(rpt // zr):
                r0 = sid * rpt + j * zr
                pltpu.sync_copy(acc_sh.at[pl.ds(r0, zr)],
                                out_hbm.at[pl.ds(r0, zr), pl.ds(qt * 128, 128)])
            plsc.subcore_barrier()
    return k


_sc_deg = _make_sc_deg(N, E_PAD)
_sc_gcn0 = _make_sc_gcn0(N, E_PAD)
_sc_att = _make_sc_att(N, E_PAD)
_sc_ea = _make_sc_ea(N, E_PAD)
_sc_passb_gat = _make_sc_passb(N, E, 4, 0)
_sc_passb_c = _make_sc_passb(N, E, 8, 0)
_sc_passb_t = _make_sc_passb(N, E, 8, 4)


def _mm_body(x_ref, w_ref, b_ref, m_ref, s_ref, e_ref, o_ref, *, act):
    x = (x_ref[...] - m_ref[...]) * s_ref[...] + e_ref[...]
    if act == "elu":
        x = jnp.where(x > 0, x, jnp.exp(jnp.minimum(x, 0.0)) - 1.0)
    o_ref[...] = jnp.dot(x, w_ref[...], preferred_element_type=jnp.float32) \
        + b_ref[...]


def _mm(x, w, b, m=None, s=None, be=None, act="none"):
    n, kdim = x.shape
    f = w.shape[1]
    if m is None:
        m = jnp.zeros((kdim,), jnp.float32)
    if s is None:
        s = jnp.ones((kdim,), jnp.float32)
    if be is None:
        be = jnp.zeros((kdim,), jnp.float32)
    grid = (n // BM, f // 128)
    return pl.pallas_call(
        functools.partial(_mm_body, act=act),
        grid=grid,
        in_specs=[pl.BlockSpec((BM, kdim), lambda i, j: (i, 0)),
                  pl.BlockSpec((kdim, 128), lambda i, j: (0, j)),
                  pl.BlockSpec((1, 128), lambda i, j: (0, j)),
                  pl.BlockSpec((1, kdim), lambda i, j: (0, 0)),
                  pl.BlockSpec((1, kdim), lambda i, j: (0, 0)),
                  pl.BlockSpec((1, kdim), lambda i, j: (0, 0))],
        out_specs=pl.BlockSpec((BM, 128), lambda i, j: (i, j)),
        out_shape=jax.ShapeDtypeStruct((n, f), jnp.float32),
    )(x, w, b.reshape(1, f), m.reshape(1, kdim), s.reshape(1, kdim),
      be.reshape(1, kdim))


def _mmq_body(x_ref, w_ref, b_ref, m_ref, s_ref, e_ref, o_ref, q_ref, *, act):
    x = (x_ref[...] - m_ref[...]) * s_ref[...] + e_ref[...]
    if act == "elu":
        x = jnp.where(x > 0, x, jnp.exp(jnp.minimum(x, 0.0)) - 1.0)
    r = jnp.dot(x, w_ref[...], preferred_element_type=jnp.float32) + b_ref[...]
    o_ref[...] = r
    q_ref[...] = r[None]


def _mmq(x, w, b, m=None, s=None, be=None, act="none"):
    n, kdim = x.shape
    f = w.shape[1]
    if m is None:
        m = jnp.zeros((kdim,), jnp.float32)
    if s is None:
        s = jnp.ones((kdim,), jnp.float32)
    if be is None:
        be = jnp.zeros((kdim,), jnp.float32)
    grid = (n // BM, f // 128)
    return pl.pallas_call(
        functools.partial(_mmq_body, act=act),
        grid=grid,
        in_specs=[pl.BlockSpec((BM, kdim), lambda i, j: (i, 0)),
                  pl.BlockSpec((kdim, 128), lambda i, j: (0, j)),
                  pl.BlockSpec((1, 128), lambda i, j: (0, j)),
                  pl.BlockSpec((1, kdim), lambda i, j: (0, 0)),
                  pl.BlockSpec((1, kdim), lambda i, j: (0, 0)),
                  pl.BlockSpec((1, kdim), lambda i, j: (0, 0))],
        out_specs=[pl.BlockSpec((BM, 128), lambda i, j: (i, j)),
                   pl.BlockSpec((1, BM, 128), lambda i, j: (j, i, 0))],
        out_shape=[jax.ShapeDtypeStruct((n, f), jnp.float32),
                   jax.ShapeDtypeStruct((f // 128, n, 128), jnp.float32)],
    )(x, w, b.reshape(1, f), m.reshape(1, kdim), s.reshape(1, kdim),
      be.reshape(1, kdim))


def _stats_body(x_ref, o_ref):
    i = pl.program_id(0)
    x = x_ref[...]
    s = jnp.sum(x, axis=0, keepdims=True)
    s2 = jnp.sum(x * x, axis=0, keepdims=True)
    blk = jnp.concatenate([s, s2, jnp.zeros((6, x.shape[1]), jnp.float32)], 0)

    @pl.when(i == 0)
    def _():
        o_ref[...] = blk

    @pl.when(i > 0)
    def _():
        o_ref[...] = o_ref[...] + blk


def _colstats(x):
    n, f = x.shape
    return pl.pallas_call(
        _stats_body,
        grid=(n // BM,),
        in_specs=[pl.BlockSpec((BM, f), lambda i: (i, 0))],
        out_specs=pl.BlockSpec((8, f), lambda i: (0, 0)),
        out_shape=jax.ShapeDtypeStruct((8, f), jnp.float32),
    )(x)


def _bn_affine(stats, g, n):
    mean = stats[0] / n
    var = stats[1] / n - mean * mean
    return mean, g / jnp.sqrt(var + EPS)


def _gcn0_ep_body(a_ref, d_ref, h_ref, b_ref, o_ref):
    deg = d_ref[0, :, 0:1] + d_ref[1, :, 0:1] + 1.0
    dis = 1.0 / jnp.sqrt(deg)
    acc = a_ref[0] + a_ref[1]
    o_ref[...] = jnp.maximum(dis * acc + dis * dis * h_ref[...] + b_ref[...],
                             0.0)


def _gcn0_ep(accp, degp, h0, b0p):
    return pl.pallas_call(
        _gcn0_ep_body,
        grid=(N // BM,),
        in_specs=[pl.BlockSpec((2, BM, 128), lambda i: (0, i, 0)),
                  pl.BlockSpec((2, BM, 128), lambda i: (0, i, 0)),
                  pl.BlockSpec((BM, 128), lambda i: (i, 0)),
                  pl.BlockSpec((1, 128), lambda i: (0, 0))],
        out_specs=pl.BlockSpec((BM, 128), lambda i: (i, 0)),
        out_shape=jax.ShapeDtypeStruct((N, 128), jnp.float32),
    )(accp, degp, h0, b0p.reshape(1, 128))


def _expand_heads(v8, bm):
    return jnp.concatenate(
        [jnp.broadcast_to(v8[:, k:k + 1], (bm, HID)) for k in range(HEADS)], 1)


def _gat_ep_body(acc_ref, dn_ref, t_ref, hw_ref, b_ref, o_ref):
    t = t_ref[...]
    es = t[:, 0:8]
    ed = t[:, 8:16]
    a = es + ed
    p_self = jnp.exp(jnp.maximum(a, 0.2 * a))
    den8 = dn_ref[0, :, 0:8] + dn_ref[1, :, 0:8] + p_self
    den = _expand_heads(den8, acc_ref.shape[0])
    ps = _expand_heads(p_self, acc_ref.shape[0])
    o_ref[...] = (acc_ref[...] + ps * hw_ref[...]) / den + b_ref[...]


def _gat_ep(acc, denp, t, hw, b):
    return pl.pallas_call(
        _gat_ep_body,
        grid=(N // BM,),
        in_specs=[pl.BlockSpec((BM, 512), lambda i: (i, 0)),
                  pl.BlockSpec((2, BM, 128), lambda i: (0, i, 0)),
                  pl.BlockSpec((BM, 128), lambda i: (i, 0)),
                  pl.BlockSpec((BM, 512), lambda i: (i, 0)),
                  pl.BlockSpec((1, 512), lambda i: (0, 0))],
        out_specs=pl.BlockSpec((BM, 512), lambda i: (i, 0)),
        out_shape=jax.ShapeDtypeStruct((N, 512), jnp.float32),
    )(acc, denp, t, hw, b.reshape(1, 512))


def _ct_pre_body(hc_ref, ht_ref, d_ref, o_ref):
    j = pl.program_id(1)
    lane = d_ref[0, :, 0:2] + d_ref[1, :, 0:2] + 1.0
    dis_c = 1.0 / jnp.sqrt(lane[:, 0:1])
    dis_t = 1.0 / jnp.sqrt(lane[:, 1:2])
    dis = jnp.where(j < 4, dis_c, dis_t)
    h = jnp.where(j < 4, hc_ref[...], ht_ref[...])
    o_ref[...] = (dis * h)[None]


def _ct_prescale(hc, ht, degp):
    return pl.pallas_call(
        _ct_pre_body,
        grid=(N // BM, 8),
        in_specs=[pl.BlockSpec((BM, 128), lambda i, j: (i, j % 4)),
                  pl.BlockSpec((BM, 128), lambda i, j: (i, j % 4)),
                  pl.BlockSpec((2, BM, 128), lambda i, j: (0, i, 0))],
        out_specs=pl.BlockSpec((1, BM, 128), lambda i, j: (j, i, 0)),
        out_shape=jax.ShapeDtypeStruct((8, N, 128), jnp.float32),
    )(hc, ht, degp)


def _ct_ep_body(ac_ref, at_ref, d_ref, hc_ref, ht_ref, bc_ref,
                bt_ref, oc_ref, ot_ref):
    lane = d_ref[0, :, 0:2] + d_ref[1, :, 0:2] + 1.0
    dis_c = 1.0 / jnp.sqrt(lane[:, 0:1])
    dis_t = 1.0 / jnp.sqrt(lane[:, 1:2])
    oc_ref[...] = dis_c * ac_ref[...] + dis_c * dis_c * hc_ref[...] + bc_ref[...]
    ot_ref[...] = dis_t * at_ref[...] + dis_t * dis_t * ht_ref[...] + bt_ref[...]


def _ct_ep(acc_c, acc_t, degp, hc, ht, bc, bt):
    return pl.pallas_call(
        _ct_ep_body,
        grid=(N // BM,),
        in_specs=[pl.BlockSpec((BM, 512), lambda i: (i, 0)),
                  pl.BlockSpec((BM, 512), lambda i: (i, 0)),
                  pl.BlockSpec((2, BM, 128), lambda i: (0, i, 0)),
                  pl.BlockSpec((BM, 512), lambda i: (i, 0)),
                  pl.BlockSpec((BM, 512), lambda i: (i, 0)),
                  pl.BlockSpec((1, 512), lambda i: (0, 0)),
                  pl.BlockSpec((1, 512), lambda i: (0, 0))],
        out_specs=[pl.BlockSpec((BM, 512), lambda i: (i, 0)),
                   pl.BlockSpec((BM, 512), lambda i: (i, 0))],
        out_shape=[jax.ShapeDtypeStruct((N, 512), jnp.float32),
                   jax.ShapeDtypeStruct((N, 512), jnp.float32)],
    )(acc_c, acc_t, degp, hc, ht,
      bc.reshape(1, 512), bt.reshape(1, 512))


def _pool_body(b_ref, c_ref, t_ref, oc_ref, ot_ref, on_ref):
    i = pl.program_id(0)
    batch = b_ref[0, 0, :]
    gi = lax.broadcasted_iota(jnp.int32, (G, BM), 0)
    oh = (gi == batch[None, :]).astype(jnp.float32)
    pc = jnp.dot(oh, c_ref[...], preferred_element_type=jnp.float32)
    pt = jnp.dot(oh, t_ref[...], preferred_element_type=jnp.float32)
    cnt = jnp.concatenate([jnp.sum(oh, axis=1, keepdims=True),
                           jnp.zeros((G, 127), jnp.float32)], 1)

    @pl.when(i == 0)
    def _():
        oc_ref[...] = pc
        ot_ref[...] = pt
        on_ref[...] = cnt

    @pl.when(i > 0)
    def _():
        oc_ref[...] = oc_ref[...] + pc
        ot_ref[...] = ot_ref[...] + pt
        on_ref[...] = on_ref[...] + cnt


def _pool(batch3, causal, trivial):
    return pl.pallas_call(
        _pool_body,
        grid=(N // BM,),
        in_specs=[pl.BlockSpec((1, 1, BM), lambda i: (i, 0, 0)),
                  pl.BlockSpec((BM, 512), lambda i: (i, 0)),
                  pl.BlockSpec((BM, 512), lambda i: (i, 0))],
        out_specs=[pl.BlockSpec((G, 512), lambda i: (0, 0)),
                   pl.BlockSpec((G, 512), lambda i: (0, 0)),
                   pl.BlockSpec((G, 128), lambda i: (0, 0))],
        out_shape=[jax.ShapeDtypeStruct((G, 512), jnp.float32),
                   jax.ShapeDtypeStruct((G, 512), jnp.float32),
                   jax.ShapeDtypeStruct((G, 128), jnp.float32)],
    )(batch3, causal, trivial)


def _bn64(x, g, be):
    m = jnp.mean(x, axis=0, keepdims=True)
    v = jnp.mean(x * x, axis=0, keepdims=True) - m * m
    return (x - m) / jnp.sqrt(v + EPS) * g + be


def _lsm(logits):
    lm = jnp.max(logits, axis=-1, keepdims=True)
    return logits - lm - jnp.log(jnp.sum(jnp.exp(logits - lm), axis=-1,
                                         keepdims=True))


def _heads_body(sc_ref, st_ref, cnt_ref, wc1_ref, bc1_ref, gc_ref, bec_ref,
                wc2_ref, bc2_ref, wt1_ref, bt1_ref, gt_ref, bet_ref,
                wt2_ref, bt2_ref, wo1_ref, bo1_ref, go_ref, beo_ref,
                wo2_ref, bo2_ref, oc_ref, ot_ref, oo_ref):
    cnt = jnp.maximum(cnt_ref[...][:, 0:1], 1.0)
    pc = sc_ref[...] / cnt
    pt = st_ref[...] / cnt

    def clf(x, w1, b1, g, be, w2, b2):
        h = jnp.maximum(_bn64(
            jnp.dot(x, w1, preferred_element_type=jnp.float32) + b1, g, be), 0.0)
        return jnp.dot(h, w2, preferred_element_type=jnp.float32) + b2

    lc = clf(pc, wc1_ref[...], bc1_ref[...], gc_ref[...], bec_ref[...],
             wc2_ref[...], bc2_ref[...])
    lt = clf(pt, wt1_ref[...], bt1_ref[...], gt_ref[...], bet_ref[...],
             wt2_ref[...], bt2_ref[...])
    comb = jnp.concatenate([pc, pt], axis=1)
    lo = clf(comb, wo1_ref[...], bo1_ref[...], go_ref[...], beo_ref[...],
             wo2_ref[...], bo2_ref[...])
    oc_ref[...] = _lsm(lc[:, 0:NC])
    ot_ref[...] = _lsm(lt[:, 0:NC])
    oo_ref[...] = _lsm(lo[:, 0:NC])


def _heads(sum_c, sum_t, cnt, Wc1, bc1, gc, bec, Wc2, bc2, Wt1, bt1, gt, bet,
           Wt2, bt2, Wo1, bo1, go, beo, Wo2, bo2):
    full = lambda shp: pl.BlockSpec(shp, lambda: tuple(0 for _ in shp))
    args = [sum_c, sum_t, cnt,
            Wc1, bc1.reshape(1, HID), gc.reshape(1, HID), bec.reshape(1, HID),
            Wc2, bc2.reshape(1, NC),
            Wt1, bt1.reshape(1, HID), gt.reshape(1, HID), bet.reshape(1, HID),
            Wt2, bt2.reshape(1, NC),
            Wo1, bo1.reshape(1, HID), go.reshape(1, HID), beo.reshape(1, HID),
            Wo2, bo2.reshape(1, NC)]
    return pl.pallas_call(
        _heads_body,
        grid=(),
        in_specs=[full(a.shape) for a in args],
        out_specs=[full((G, NC))] * 3,
        out_shape=[jax.ShapeDtypeStruct((G, NC), jnp.float32)] * 3,
    )(*args)


def _att_compose(W, a_s, a_d):
    ces = (jnp.eye(HEADS, dtype=jnp.float32)[:, None, :]
           * a_s[:, :, None]).reshape(512, HEADS)
    ced = (jnp.eye(HEADS, dtype=jnp.float32)[:, None, :]
           * a_d[:, :, None]).reshape(512, HEADS)
    C = jnp.concatenate([ces, ced], axis=1)
    B = W @ C
    return jnp.pad(B, ((0, 0), (0, 112)))


def kernel(x, edge_index, batch, W0, b0, g0, be0, W1, as1, ad1, b1, g1, be1,
           W2, as2, ad2, b2, g2, be2, Wna, bna, Wea, bea, Wc, bc, Wt, bt,
           Wc1, bc1, gc, bec, Wc2, bc2, Wt1, bt1, gt, bet, Wt2, bt2,
           Wo1, bo1, go, beo, Wo2, bo2):
    src = jnp.concatenate(
        [edge_index[0].astype(jnp.int32),
         jnp.zeros((E_PAD - E,), jnp.int32)])
    dst = jnp.concatenate(
        [edge_index[1].astype(jnp.int32),
         jnp.full((E_PAD - E,), N, jnp.int32)])

    degp = _sc_deg(dst)[:, :N]
    W0p = jnp.pad(W0, ((0, 0), (0, 128 - HID)))
    h0 = _mm(x, W0p, jnp.zeros((128,), jnp.float32))
    g0tab = _gcn0_pre(h0, degp)
    accp = _sc_gcn0(g0tab, src, dst)[:, :N]
    b0p = jnp.pad(b0, (0, 128 - HID))
    r0 = _gcn0_ep(accp, degp, h0, b0p)
    st0 = _colstats(r0)
    g0p = jnp.pad(g0, (0, 128 - HID))
    be0p = jnp.pad(be0, (0, 128 - HID))
    m0, s0 = _bn_affine(st0, g0p, N)

    W1p = jnp.pad(W1, ((0, 128 - HID), (0, 0)))
    hw1, h4_1 = _mmq(r0, W1p, jnp.zeros((512,), jnp.float32), m0, s0, be0p)
    B1 = jnp.pad(_att_compose(W1, as1, ad1), ((0, 128 - HID), (0, 0)))
    t1 = _mm(r0, B1, jnp.zeros((128,), jnp.float32), m0, s0, be0p)
    p1, denp1 = _sc_att(t1, src, dst)
    acc1 = _sc_passb_gat(h4_1, src, dst, p1)[:N]
    out1 = _gat_ep(acc1, denp1[:, :N], t1, hw1, b1)
    st1 = _colstats(out1)
    m1, s1 = _bn_affine(st1, g1, N)

    hw2, h4_2 = _mmq(out1, W2, jnp.zeros((512,), jnp.float32), m1, s1, be1,
                     act="elu")
    B2 = _att_compose(W2, as2, ad2)
    t2 = _mm(out1, B2, jnp.zeros((128,), jnp.float32), m1, s1, be1, act="elu")
    p2, denp2 = _sc_att(t2, src, dst)
    acc2 = _sc_passb_gat(h4_2, src, dst, p2)[:N]
    out2 = _gat_ep(acc2, denp2[:, :N], t2, hw2, b2)
    st2 = _colstats(out2)
    m2, s2 = _bn_affine(st2, g2, N)

    hc = _mm(out2, Wc, jnp.zeros((512,), jnp.float32), m2, s2, be2, act="elu")
    ht = _mm(out2, Wt, jnp.zeros((512,), jnp.float32), m2, s2, be2, act="elu")
    B3 = jnp.pad(jnp.concatenate([Wea[:D], Wea[D:]], axis=1),
                 ((0, 0), (0, 124)))
    b3 = jnp.pad(bea, (0, 126))
    t3 = _mm(out2, B3, b3, m2, s2, be2, act="elu")
    ea, degct = _sc_ea(t3, src, dst)
    degct = degct[:, :N]
    hq8 = _ct_prescale(hc, ht, degct)
    acc_c = _sc_passb_c(hq8, src, dst, ea)[:N]
    acc_t = _sc_passb_t(hq8, src, dst, ea)[:N]
    causal, trivial = _ct_ep(acc_c, acc_t, degct, hc, ht, bc, bt)

    batch3 = batch.astype(jnp.int32).reshape(N // BM, 1, BM)
    sum_c, sum_t, cnt = _pool(batch3, causal, trivial)
    oc, ot, oco = _heads(sum_c, sum_t, cnt, Wc1, bc1, gc, bec, Wc2, bc2,
                         Wt1, bt1, gt, bet, Wt2, bt2, Wo1, bo1, go, beo,
                         Wo2, bo2)
    return (oc, ot, oco)


def _gcn0_pre_body(h_ref, d_ref, o_ref):
    deg = d_ref[0, :, 0:1] + d_ref[1, :, 0:1] + 1.0
    o_ref[...] = h_ref[...] / jnp.sqrt(deg)


def _gcn0_pre(h0, degp):
    return pl.pallas_call(
        _gcn0_pre_body,
        grid=(N // BM,),
        in_specs=[pl.BlockSpec((BM, 128), lambda i: (i, 0)),
                  pl.BlockSpec((2, BM, 128), lambda i: (0, i, 0))],
        out_specs=pl.BlockSpec((BM, 128), lambda i: (i, 0)),
        out_shape=jax.ShapeDtypeStruct((N, 128), jnp.float32),
    )(h0, degp)


# speedup vs baseline: 12.4365x; 1.3886x over previous
import functools

import jax
import jax.numpy as jnp
from jax import lax
from jax.experimental import pallas as pl
from jax.experimental.pallas import tpu as pltpu
from jax.experimental.pallas import tpu_sc as plsc

N = 10000
E = 160000
F_IN = 128
HID = 64
HEADS = 8
D = HID * HEADS
G = 64
NC = 10
EPS = 1e-5

BM = 1000
NW = 32
E_PAD = E + 256

_mesh = plsc.VectorSubcoreMesh(core_axis_name="c", subcore_axis_name="s")


def _dyng(v, idx):
    return lax.gather(
        v, idx[:, None],
        lax.GatherDimensionNumbers(offset_dims=(), collapsed_slice_dims=(0,),
                                   start_index_map=(0,)),
        (1,), mode=lax.GatherScatterMode.PROMISE_IN_BOUNDS)


def _splat(v, lane):
    return _dyng(v, jnp.full((16,), lane, jnp.int32))


def _zero_rows(z_v, rows, cols):
    def zr(i, _):
        for k in range(cols // 16):
            z_v[i, pl.ds(k * 16, 16)] = jnp.zeros((16,), jnp.float32)
        return 0
    lax.fori_loop(0, rows, zr, 0)


def _make_sc_deg(n, e):
    epw = e // NW
    npad = -(-n // 1280) * 1280
    rpt = npad // 16
    zr = 32
    nb, tail = divmod(epw, 16)

    @functools.partial(
        pl.kernel,
        out_type=jax.ShapeDtypeStruct((2, npad, 128), jnp.float32),
        mesh=_mesh,
        scratch_types=[pltpu.VMEM((epw,), jnp.int32),
                       pltpu.VMEM((16, 128), jnp.float32),
                       pltpu.VMEM((zr, 128), jnp.float32),
                       pltpu.VMEM_SHARED((npad, 128), jnp.float32),
                       pltpu.SemaphoreType.DMA])
    def k(dst_hbm, out_hbm, dst_v, one_v, z_v, acc_sh, sem):
        cid = lax.axis_index("c")
        sid = lax.axis_index("s")
        wid = cid * 16 + sid
        _zero_rows(z_v, zr, 128)
        for j in range(rpt // zr):
            pltpu.sync_copy(z_v, acc_sh.at[pl.ds(sid * rpt + j * zr, zr)])
        iota = lax.iota(jnp.int32, 16)
        ones0 = jnp.where(iota == 0, 1.0, 0.0)
        for i in range(16):
            one_v[i, pl.ds(0, 16)] = ones0
            for kk in range(1, 8):
                one_v[i, pl.ds(kk * 16, 16)] = jnp.zeros((16,), jnp.float32)
        plsc.subcore_barrier()
        pltpu.sync_copy(dst_hbm.at[pl.ds(wid * epw, epw)], dst_v)

        def blk(j, _):
            didx = dst_v[pl.ds(j * 16, 16)]
            pltpu.sync_copy(one_v, acc_sh.at[didx], add=True)
            return 0
        lax.fori_loop(0, nb, blk, 0)
        if tail:
            didx = dst_v[pl.ds(nb * 16, tail)]
            pltpu.sync_copy(one_v.at[pl.ds(0, tail)], acc_sh.at[didx], add=True)
        plsc.subcore_barrier()
        for j in range(rpt // zr):
            r0 = sid * rpt + j * zr
            pltpu.sync_copy(acc_sh.at[pl.ds(r0, zr)], out_hbm.at[cid, pl.ds(r0, zr)])
    return k


def _make_sc_gcn0(n, e):
    epw = e // NW
    npad = -(-n // 1280) * 1280
    rpt = npad // 16
    zr = 32
    nb, tail = divmod(epw, 16)

    @functools.partial(
        pl.kernel,
        out_type=jax.ShapeDtypeStruct((2, npad, 128), jnp.float32),
        mesh=_mesh,
        scratch_types=[pltpu.VMEM((epw,), jnp.int32),
                       pltpu.VMEM((epw,), jnp.int32),
                       pltpu.VMEM((16, 128), jnp.float32),
                       pltpu.VMEM((zr, 128), jnp.float32),
                       pltpu.VMEM_SHARED((npad, 128), jnp.float32),
                       pltpu.SemaphoreType.DMA])
    def k(tab_hbm, src_hbm, dst_hbm, out_hbm, src_v, dst_v, rows_v, z_v,
          acc_sh, sem):
        cid = lax.axis_index("c")
        sid = lax.axis_index("s")
        wid = cid * 16 + sid
        _zero_rows(z_v, zr, 128)
        for j in range(rpt // zr):
            pltpu.sync_copy(z_v, acc_sh.at[pl.ds(sid * rpt + j * zr, zr)])
        plsc.subcore_barrier()
        pltpu.sync_copy(src_hbm.at[pl.ds(wid * epw, epw)], src_v)
        pltpu.sync_copy(dst_hbm.at[pl.ds(wid * epw, epw)], dst_v)

        def blk(j, _):
            sidx = src_v[pl.ds(j * 16, 16)]
            pltpu.async_copy(tab_hbm.at[sidx], rows_v, sem).wait()
            didx = dst_v[pl.ds(j * 16, 16)]
            pltpu.sync_copy(rows_v, acc_sh.at[didx], add=True)
            return 0
        lax.fori_loop(0, nb, blk, 0)
        if tail:
            sidx = src_v[pl.ds(nb * 16, tail)]
            pltpu.async_copy(tab_hbm.at[sidx], rows_v.at[pl.ds(0, tail)], sem).wait()
            didx = dst_v[pl.ds(nb * 16, tail)]
            pltpu.sync_copy(rows_v.at[pl.ds(0, tail)], acc_sh.at[didx], add=True)
        plsc.subcore_barrier()
        for j in range(rpt // zr):
            r0 = sid * rpt + j * zr
            pltpu.sync_copy(acc_sh.at[pl.ds(r0, zr)], out_hbm.at[cid, pl.ds(r0, zr)])
    return k


def _make_sc_att(n, e):
    epw = e // NW
    npad = -(-n // 1280) * 1280
    rpt = npad // 16
    zr = 32
    nb, tail = divmod(epw, 16)

    @functools.partial(
        pl.kernel,
        out_type=(jax.ShapeDtypeStruct((e * 16,), jnp.float32),
                  jax.ShapeDtypeStruct((2, npad, 128), jnp.float32)),
        mesh=_mesh,
        scratch_types=[pltpu.VMEM((epw,), jnp.int32),
                       pltpu.VMEM((epw,), jnp.int32),
                       pltpu.VMEM((16, 128), jnp.float32),
                       pltpu.VMEM((16, 128), jnp.float32),
                       pltpu.VMEM((256,), jnp.float32),
                       pltpu.VMEM((16, 128), jnp.float32),
                       pltpu.VMEM((zr, 128), jnp.float32),
                       pltpu.VMEM_SHARED((npad, 128), jnp.float32),
                       pltpu.SemaphoreType.DMA,
                       pltpu.SemaphoreType.DMA])
    def k(t_hbm, src_hbm, dst_hbm, p_hbm, den_hbm,
          src_v, dst_v, rs_v, rd_v, pb_v, db_v, z_v, acc_sh, sem, sem2):
        cid = lax.axis_index("c")
        sid = lax.axis_index("s")
        wid = cid * 16 + sid
        base = wid * epw
        _zero_rows(z_v, zr, 128)
        for j in range(rpt // zr):
            pltpu.sync_copy(z_v, acc_sh.at[pl.ds(sid * rpt + j * zr, zr)])
        _zero_rows(db_v, 16, 128)
        plsc.subcore_barrier()
        pltpu.sync_copy(src_hbm.at[pl.ds(base, epw)], src_v)
        pltpu.sync_copy(dst_hbm.at[pl.ds(base, epw)], dst_v)
        iota = lax.iota(jnp.int32, 16)
        rot = iota % 8 + 8

        def do_blk(j, nrow):
            sidx = src_v[pl.ds(j * 16, nrow)]
            didx = dst_v[pl.ds(j * 16, nrow)]
            c1 = pltpu.async_copy(t_hbm.at[sidx], rs_v.at[pl.ds(0, nrow)], sem)
            c2 = pltpu.async_copy(t_hbm.at[didx], rd_v.at[pl.ds(0, nrow)], sem2)
            c1.wait()
            c2.wait()
            for ee in range(nrow):
                es = rs_v[ee, pl.ds(0, 16)]
                ed = _dyng(rd_v[ee, pl.ds(0, 16)], rot)
                a = es + ed
                a = jnp.maximum(a, 0.2 * a)
                p = jnp.where(iota < 8, jnp.exp(a), 0.0)
                pb_v[pl.ds(ee * 16, 16)] = p
                db_v[ee, pl.ds(0, 16)] = p
            pltpu.sync_copy(pb_v,
                            p_hbm.at[pl.ds((base + j * 16) * 16, 256)])
            pltpu.sync_copy(db_v.at[pl.ds(0, nrow)], acc_sh.at[didx], add=True)

        def blk(j, _):
            do_blk(j, 16)
            return 0
        lax.fori_loop(0, nb, blk, 0)
        if tail:
            do_blk(nb, tail)
        plsc.subcore_barrier()
        for j in range(rpt // zr):
            r0 = sid * rpt + j * zr
            pltpu.sync_copy(acc_sh.at[pl.ds(r0, zr)], den_hbm.at[cid, pl.ds(r0, zr)])
    return k


def _make_sc_ea(n, e):
    epw = e // NW
    npad = -(-n // 1280) * 1280
    rpt = npad // 16
    zr = 32
    nb, tail = divmod(epw, 16)

    @functools.partial(
        pl.kernel,
        out_type=(jax.ShapeDtypeStruct((e * 16,), jnp.float32),
                  jax.ShapeDtypeStruct((2, npad, 128), jnp.float32)),
        mesh=_mesh,
        scratch_types=[pltpu.VMEM((epw,), jnp.int32),
                       pltpu.VMEM((epw,), jnp.int32),
                       pltpu.VMEM((16, 128), jnp.float32),
                       pltpu.VMEM((16, 128), jnp.float32),
                       pltpu.VMEM((256,), jnp.float32),
                       pltpu.VMEM((16, 128), jnp.float32),
                       pltpu.VMEM((zr, 128), jnp.float32),
                       pltpu.VMEM_SHARED((npad, 128), jnp.float32),
                       pltpu.SemaphoreType.DMA,
                       pltpu.SemaphoreType.DMA])
    def k(t_hbm, src_hbm, dst_hbm, ea_hbm, deg_hbm,
          src_v, dst_v, rs_v, rd_v, eb_v, db_v, z_v, acc_sh, sem, sem2):
        cid = lax.axis_index("c")
        sid = lax.axis_index("s")
        wid = cid * 16 + sid
        base = wid * epw
        _zero_rows(z_v, zr, 128)
        for j in range(rpt // zr):
            pltpu.sync_copy(z_v, acc_sh.at[pl.ds(sid * rpt + j * zr, zr)])
        _zero_rows(db_v, 16, 128)
        plsc.subcore_barrier()
        pltpu.sync_copy(src_hbm.at[pl.ds(base, epw)], src_v)
        pltpu.sync_copy(dst_hbm.at[pl.ds(base, epw)], dst_v)
        iota = lax.iota(jnp.int32, 16)
        rot2 = iota % 2 + 2

        def do_blk(j, nrow):
            sidx = src_v[pl.ds(j * 16, nrow)]
            didx = dst_v[pl.ds(j * 16, nrow)]
            c1 = pltpu.async_copy(t_hbm.at[sidx], rs_v.at[pl.ds(0, nrow)], sem)
            c2 = pltpu.async_copy(t_hbm.at[didx], rd_v.at[pl.ds(0, nrow)], sem2)
            c1.wait()
            c2.wait()
            for ee in range(nrow):
                l = rs_v[ee, pl.ds(0, 16)] + _dyng(rd_v[ee, pl.ds(0, 16)], rot2)
                ldiff = _splat(l, 1) - _splat(l, 0)
                ea0 = 1.0 / (1.0 + jnp.exp(ldiff))
                ea1 = 1.0 - ea0
                eb_v[pl.ds(ee * 16, 16)] = jnp.where(iota < 8, ea0, ea1)
                db_v[ee, pl.ds(0, 16)] = jnp.where(
                    iota == 0, ea0, jnp.where(iota == 1, ea1, 0.0))
            pltpu.sync_copy(eb_v,
                            ea_hbm.at[pl.ds((base + j * 16) * 16, 256)])
            pltpu.sync_copy(db_v.at[pl.ds(0, nrow)], acc_sh.at[didx], add=True)

        def blk(j, _):
            do_blk(j, 16)
            return 0
        lax.fori_loop(0, nb, blk, 0)
        if tail:
            do_blk(nb, tail)
        plsc.subcore_barrier()
        for j in range(rpt // zr):
            r0 = sid * rpt + j * zr
            pltpu.sync_copy(acc_sh.at[pl.ds(r0, zr)], deg_hbm.at[cid, pl.ds(r0, zr)])
    return k


def _make_sc_passb(n, e, nq, qbase):
    ept = e // 16
    grp = 2000
    ngrp = ept // grp
    nbg = grp // 16
    npad = -(-n // 1280) * 1280
    rpt = npad // 16
    zr = 32

    nb2, blkrem = divmod(nbg, 2)

    @functools.partial(
        pl.kernel,
        out_type=jax.ShapeDtypeStruct((npad, 512), jnp.float32),
        mesh=_mesh,
        scratch_types=[pltpu.VMEM((grp,), jnp.int32),
                       pltpu.VMEM((grp,), jnp.int32),
                       pltpu.VMEM((512,), jnp.float32),
                       pltpu.VMEM((16, 128), jnp.float32),
                       pltpu.VMEM((16, 128), jnp.float32),
                       pltpu.VMEM((16, 128), jnp.float32),
                       pltpu.VMEM((16, 128), jnp.float32),
                       pltpu.VMEM((zr, 128), jnp.float32),
                       pltpu.VMEM_SHARED((npad, 128), jnp.float32),
                       pltpu.SemaphoreType.DMA,
                       pltpu.SemaphoreType.DMA,
                       pltpu.SemaphoreType.DMA,
                       pltpu.SemaphoreType.DMA])
    def k(h_hbm, src_hbm, dst_hbm, p_hbm, out_hbm,
          src_v, dst_v, p_v, rows_a, rows_b, ob_a, ob_b, z_v, acc_sh,
          semg0, semg1, sems0, sems1):
        cid = lax.axis_index("c")
        sid = lax.axis_index("s")
        ebase = sid * ept
        for kq in range(2):
            qt = cid * 2 + kq
            q = qbase + qt
            lane0 = jnp.full((16,), 2 * q, jnp.int32)
            lane1 = lane0 + 1
            _zero_rows(z_v, zr, 128)
            for j in range(rpt // zr):
                pltpu.sync_copy(z_v, acc_sh.at[pl.ds(sid * rpt + j * zr, zr)])
            plsc.subcore_barrier()

            def compute(rows_v, ob_v, poff):
                for ee in range(16):
                    pr = p_v[pl.ds(poff + ee * 16, 16)]
                    w0 = _dyng(pr, lane0)
                    w1 = _dyng(pr, lane1)
                    for v in range(4):
                        ob_v[ee, pl.ds(v * 16, 16)] = (
                            rows_v[ee, pl.ds(v * 16, 16)] * w0)
                    for v in range(4, 8):
                        ob_v[ee, pl.ds(v * 16, 16)] = (
                            rows_v[ee, pl.ds(v * 16, 16)] * w1)

            def grp_body(g, _):
                goff = ebase + g * grp
                pltpu.sync_copy(src_hbm.at[pl.ds(goff, grp)], src_v)
                pltpu.sync_copy(dst_hbm.at[pl.ds(goff, grp)], dst_v)

                def blk2(j2, _):
                    j = j2 * 2
                    c0 = pltpu.async_copy(
                        h_hbm.at[q].at[src_v[pl.ds(j * 16, 16)]], rows_a,
                        semg0)
                    c1 = pltpu.async_copy(
                        h_hbm.at[q].at[src_v[pl.ds(j * 16 + 16, 16)]], rows_b,
                        semg1)
                    pltpu.sync_copy(p_hbm.at[pl.ds((goff + j * 16) * 16, 512)],
                                    p_v)
                    c0.wait()
                    compute(rows_a, ob_a, 0)
                    s0 = pltpu.async_copy(
                        ob_a, acc_sh.at[dst_v[pl.ds(j * 16, 16)]], sems0,
                        add=True)
                    c1.wait()
                    compute(rows_b, ob_b, 256)
                    s1 = pltpu.async_copy(
                        ob_b, acc_sh.at[dst_v[pl.ds(j * 16 + 16, 16)]], sems1,
                        add=True)
                    s0.wait()
                    s1.wait()
                    return 0
                lax.fori_loop(0, nb2, blk2, 0)
                if blkrem:
                    j = nb2 * 2
                    cg = pltpu.async_copy(
                        h_hbm.at[q].at[src_v[pl.ds(j * 16, 16)]], rows_a,
                        semg0)
                    pltpu.sync_copy(p_hbm.at[pl.ds((goff + j * 16) * 16, 256)],
                                    p_v.at[pl.ds(0, 256)])
                    cg.wait()
                    compute(rows_a, ob_a, 0)
                    pltpu.sync_copy(ob_a, acc_sh.at[dst_v[pl.ds(j * 16, 16)]],
                                    add=True)
                return 0
            lax.fori_loop(0, ngrp, grp_body, 0)
            plsc.subcore_barrier()
            for j in range(rpt // zr):
                r0 = sid * rpt + j * zr
                pltpu.sync_copy(acc_sh.at[pl.ds(r0, zr)],
                                out_hbm.at[pl.ds(r0, zr), pl.ds(qt * 128, 128)])
            plsc.subcore_barrier()
    return k


_sc_deg = _make_sc_deg(N, E_PAD)
_sc_gcn0 = _make_sc_gcn0(N, E_PAD)
_sc_att = _make_sc_att(N, E_PAD)
_sc_ea = _make_sc_ea(N, E_PAD)
_sc_passb_gat = _make_sc_passb(N, E, 4, 0)
_sc_passb_c = _make_sc_passb(N, E, 8, 0)
_sc_passb_t = _make_sc_passb(N, E, 8, 4)


def _mm_body(x_ref, w_ref, b_ref, m_ref, s_ref, e_ref, o_ref, *, act):
    x = (x_ref[...] - m_ref[...]) * s_ref[...] + e_ref[...]
    if act == "elu":
        x = jnp.where(x > 0, x, jnp.exp(jnp.minimum(x, 0.0)) - 1.0)
    o_ref[...] = jnp.dot(x, w_ref[...], preferred_element_type=jnp.float32) \
        + b_ref[...]


def _mm(x, w, b, m=None, s=None, be=None, act="none"):
    n, kdim = x.shape
    f = w.shape[1]
    if m is None:
        m = jnp.zeros((kdim,), jnp.float32)
    if s is None:
        s = jnp.ones((kdim,), jnp.float32)
    if be is None:
        be = jnp.zeros((kdim,), jnp.float32)
    grid = (n // BM, f // 128)
    return pl.pallas_call(
        functools.partial(_mm_body, act=act),
        grid=grid,
        in_specs=[pl.BlockSpec((BM, kdim), lambda i, j: (i, 0)),
                  pl.BlockSpec((kdim, 128), lambda i, j: (0, j)),
                  pl.BlockSpec((1, 128), lambda i, j: (0, j)),
                  pl.BlockSpec((1, kdim), lambda i, j: (0, 0)),
                  pl.BlockSpec((1, kdim), lambda i, j: (0, 0)),
                  pl.BlockSpec((1, kdim), lambda i, j: (0, 0))],
        out_specs=pl.BlockSpec((BM, 128), lambda i, j: (i, j)),
        out_shape=jax.ShapeDtypeStruct((n, f), jnp.float32),
    )(x, w, b.reshape(1, f), m.reshape(1, kdim), s.reshape(1, kdim),
      be.reshape(1, kdim))


def _mmq_body(x_ref, w_ref, b_ref, m_ref, s_ref, e_ref, o_ref, q_ref, *, act):
    x = (x_ref[...] - m_ref[...]) * s_ref[...] + e_ref[...]
    if act == "elu":
        x = jnp.where(x > 0, x, jnp.exp(jnp.minimum(x, 0.0)) - 1.0)
    r = jnp.dot(x, w_ref[...], preferred_element_type=jnp.float32) + b_ref[...]
    o_ref[...] = r
    q_ref[...] = r[None]


def _mmq(x, w, b, m=None, s=None, be=None, act="none"):
    n, kdim = x.shape
    f = w.shape[1]
    if m is None:
        m = jnp.zeros((kdim,), jnp.float32)
    if s is None:
        s = jnp.ones((kdim,), jnp.float32)
    if be is None:
        be = jnp.zeros((kdim,), jnp.float32)
    grid = (n // BM, f // 128)
    return pl.pallas_call(
        functools.partial(_mmq_body, act=act),
        grid=grid,
        in_specs=[pl.BlockSpec((BM, kdim), lambda i, j: (i, 0)),
                  pl.BlockSpec((kdim, 128), lambda i, j: (0, j)),
                  pl.BlockSpec((1, 128), lambda i, j: (0, j)),
                  pl.BlockSpec((1, kdim), lambda i, j: (0, 0)),
                  pl.BlockSpec((1, kdim), lambda i, j: (0, 0)),
                  pl.BlockSpec((1, kdim), lambda i, j: (0, 0))],
        out_specs=[pl.BlockSpec((BM, 128), lambda i, j: (i, j)),
                   pl.BlockSpec((1, BM, 128), lambda i, j: (j, i, 0))],
        out_shape=[jax.ShapeDtypeStruct((n, f), jnp.float32),
                   jax.ShapeDtypeStruct((f // 128, n, 128), jnp.float32)],
    )(x, w, b.reshape(1, f), m.reshape(1, kdim), s.reshape(1, kdim),
      be.reshape(1, kdim))


def _stats_body(x_ref, o_ref):
    i = pl.program_id(0)
    x = x_ref[...]
    s = jnp.sum(x, axis=0, keepdims=True)
    s2 = jnp.sum(x * x, axis=0, keepdims=True)
    blk = jnp.concatenate([s, s2, jnp.zeros((6, x.shape[1]), jnp.float32)], 0)

    @pl.when(i == 0)
    def _():
        o_ref[...] = blk

    @pl.when(i > 0)
    def _():
        o_ref[...] = o_ref[...] + blk


def _colstats(x):
    n, f = x.shape
    return pl.pallas_call(
        _stats_body,
        grid=(n // BM,),
        in_specs=[pl.BlockSpec((BM, f), lambda i: (i, 0))],
        out_specs=pl.BlockSpec((8, f), lambda i: (0, 0)),
        out_shape=jax.ShapeDtypeStruct((8, f), jnp.float32),
    )(x)


def _bn_affine(stats, g, n):
    mean = stats[0] / n
    var = stats[1] / n - mean * mean
    return mean, g / jnp.sqrt(var + EPS)


def _gcn0_ep_body(a_ref, d_ref, h_ref, b_ref, o_ref):
    deg = d_ref[0, :, 0:1] + d_ref[1, :, 0:1] + 1.0
    dis = 1.0 / jnp.sqrt(deg)
    acc = a_ref[0] + a_ref[1]
    o_ref[...] = jnp.maximum(dis * acc + dis * dis * h_ref[...] + b_ref[...],
                             0.0)


def _gcn0_ep(accp, degp, h0, b0p):
    return pl.pallas_call(
        _gcn0_ep_body,
        grid=(N // BM,),
        in_specs=[pl.BlockSpec((2, BM, 128), lambda i: (0, i, 0)),
                  pl.BlockSpec((2, BM, 128), lambda i: (0, i, 0)),
                  pl.BlockSpec((BM, 128), lambda i: (i, 0)),
                  pl.BlockSpec((1, 128), lambda i: (0, 0))],
        out_specs=pl.BlockSpec((BM, 128), lambda i: (i, 0)),
        out_shape=jax.ShapeDtypeStruct((N, 128), jnp.float32),
    )(accp, degp, h0, b0p.reshape(1, 128))


def _expand_heads(v8, bm):
    return jnp.concatenate(
        [jnp.broadcast_to(v8[:, k:k + 1], (bm, HID)) for k in range(HEADS)], 1)


def _gat_ep_body(acc_ref, dn_ref, t_ref, hw_ref, b_ref, o_ref):
    t = t_ref[...]
    es = t[:, 0:8]
    ed = t[:, 8:16]
    a = es + ed
    p_self = jnp.exp(jnp.maximum(a, 0.2 * a))
    den8 = dn_ref[0, :, 0:8] + dn_ref[1, :, 0:8] + p_self
    den = _expand_heads(den8, acc_ref.shape[0])
    ps = _expand_heads(p_self, acc_ref.shape[0])
    o_ref[...] = (acc_ref[...] + ps * hw_ref[...]) / den + b_ref[...]


def _gat_ep(acc, denp, t, hw, b):
    return pl.pallas_call(
        _gat_ep_body,
        grid=(N // BM,),
        in_specs=[pl.BlockSpec((BM, 512), lambda i: (i, 0)),
                  pl.BlockSpec((2, BM, 128), lambda i: (0, i, 0)),
                  pl.BlockSpec((BM, 128), lambda i: (i, 0)),
                  pl.BlockSpec((BM, 512), lambda i: (i, 0)),
                  pl.BlockSpec((1, 512), lambda i: (0, 0))],
        out_specs=pl.BlockSpec((BM, 512), lambda i: (i, 0)),
        out_shape=jax.ShapeDtypeStruct((N, 512), jnp.float32),
    )(acc, denp, t, hw, b.reshape(1, 512))


def _ct_pre_body(hc_ref, ht_ref, d_ref, o_ref):
    j = pl.program_id(1)
    lane = d_ref[0, :, 0:2] + d_ref[1, :, 0:2] + 1.0
    dis_c = 1.0 / jnp.sqrt(lane[:, 0:1])
    dis_t = 1.0 / jnp.sqrt(lane[:, 1:2])
    dis = jnp.where(j < 4, dis_c, dis_t)
    h = jnp.where(j < 4, hc_ref[...], ht_ref[...])
    o_ref[...] = (dis * h)[None]


def _ct_prescale(hc, ht, degp):
    return pl.pallas_call(
        _ct_pre_body,
        grid=(N // BM, 8),
        in_specs=[pl.BlockSpec((BM, 128), lambda i, j: (i, j % 4)),
                  pl.BlockSpec((BM, 128), lambda i, j: (i, j % 4)),
                  pl.BlockSpec((2, BM, 128), lambda i, j: (0, i, 0))],
        out_specs=pl.BlockSpec((1, BM, 128), lambda i, j: (j, i, 0)),
        out_shape=jax.ShapeDtypeStruct((8, N, 128), jnp.float32),
    )(hc, ht, degp)


def _ct_ep_body(ac_ref, at_ref, d_ref, hc_ref, ht_ref, bc_ref,
                bt_ref, oc_ref, ot_ref):
    lane = d_ref[0, :, 0:2] + d_ref[1, :, 0:2] + 1.0
    dis_c = 1.0 / jnp.sqrt(lane[:, 0:1])
    dis_t = 1.0 / jnp.sqrt(lane[:, 1:2])
    oc_ref[...] = dis_c * ac_ref[...] + dis_c * dis_c * hc_ref[...] + bc_ref[...]
    ot_ref[...] = dis_t * at_ref[...] + dis_t * dis_t * ht_ref[...] + bt_ref[...]


def _ct_ep(acc_c, acc_t, degp, hc, ht, bc, bt):
    return pl.pallas_call(
        _ct_ep_body,
        grid=(N // BM,),
        in_specs=[pl.BlockSpec((BM, 512), lambda i: (i, 0)),
                  pl.BlockSpec((BM, 512), lambda i: (i, 0)),
                  pl.BlockSpec((2, BM, 128), lambda i: (0, i, 0)),
                  pl.BlockSpec((BM, 512), lambda i: (i, 0)),
                  pl.BlockSpec((BM, 512), lambda i: (i, 0)),
                  pl.BlockSpec((1, 512), lambda i: (0, 0)),
                  pl.BlockSpec((1, 512), lambda i: (0, 0))],
        out_specs=[pl.BlockSpec((BM, 512), lambda i: (i, 0)),
                   pl.BlockSpec((BM, 512), lambda i: (i, 0))],
        out_shape=[jax.ShapeDtypeStruct((N, 512), jnp.float32),
                   jax.ShapeDtypeStruct((N, 512), jnp.float32)],
    )(acc_c, acc_t, degp, hc, ht,
      bc.reshape(1, 512), bt.reshape(1, 512))


def _pool_body(b_ref, c_ref, t_ref, oc_ref, ot_ref, on_ref):
    i = pl.program_id(0)
    batch = b_ref[0, 0, :]
    gi = lax.broadcasted_iota(jnp.int32, (G, BM), 0)
    oh = (gi == batch[None, :]).astype(jnp.float32)
    pc = jnp.dot(oh, c_ref[...], preferred_element_type=jnp.float32)
    pt = jnp.dot(oh, t_ref[...], preferred_element_type=jnp.float32)
    cnt = jnp.concatenate([jnp.sum(oh, axis=1, keepdims=True),
                           jnp.zeros((G, 127), jnp.float32)], 1)

    @pl.when(i == 0)
    def _():
        oc_ref[...] = pc
        ot_ref[...] = pt
        on_ref[...] = cnt

    @pl.when(i > 0)
    def _():
        oc_ref[...] = oc_ref[...] + pc
        ot_ref[...] = ot_ref[...] + pt
        on_ref[...] = on_ref[...] + cnt


def _pool(batch3, causal, trivial):
    return pl.pallas_call(
        _pool_body,
        grid=(N // BM,),
        in_specs=[pl.BlockSpec((1, 1, BM), lambda i: (i, 0, 0)),
                  pl.BlockSpec((BM, 512), lambda i: (i, 0)),
                  pl.BlockSpec((BM, 512), lambda i: (i, 0))],
        out_specs=[pl.BlockSpec((G, 512), lambda i: (0, 0)),
                   pl.BlockSpec((G, 512), lambda i: (0, 0)),
                   pl.BlockSpec((G, 128), lambda i: (0, 0))],
        out_shape=[jax.ShapeDtypeStruct((G, 512), jnp.float32),
                   jax.ShapeDtypeStruct((G, 512), jnp.float32),
                   jax.ShapeDtypeStruct((G, 128), jnp.float32)],
    )(batch3, causal, trivial)


def _bn64(x, g, be):
    m = jnp.mean(x, axis=0, keepdims=True)
    v = jnp.mean(x * x, axis=0, keepdims=True) - m * m
    return (x - m) / jnp.sqrt(v + EPS) * g + be


def _lsm(logits):
    lm = jnp.max(logits, axis=-1, keepdims=True)
    return logits - lm - jnp.log(jnp.sum(jnp.exp(logits - lm), axis=-1,
                                         keepdims=True))


def _heads_body(sc_ref, st_ref, cnt_ref, wc1_ref, bc1_ref, gc_ref, bec_ref,
                wc2_ref, bc2_ref, wt1_ref, bt1_ref, gt_ref, bet_ref,
                wt2_ref, bt2_ref, wo1_ref, bo1_ref, go_ref, beo_ref,
                wo2_ref, bo2_ref, oc_ref, ot_ref, oo_ref):
    cnt = jnp.maximum(cnt_ref[...][:, 0:1], 1.0)
    pc = sc_ref[...] / cnt
    pt = st_ref[...] / cnt

    def clf(x, w1, b1, g, be, w2, b2):
        h = jnp.maximum(_bn64(
            jnp.dot(x, w1, preferred_element_type=jnp.float32) + b1, g, be), 0.0)
        return jnp.dot(h, w2, preferred_element_type=jnp.float32) + b2

    lc = clf(pc, wc1_ref[...], bc1_ref[...], gc_ref[...], bec_ref[...],
             wc2_ref[...], bc2_ref[...])
    lt = clf(pt, wt1_ref[...], bt1_ref[...], gt_ref[...], bet_ref[...],
             wt2_ref[...], bt2_ref[...])
    comb = jnp.concatenate([pc, pt], axis=1)
    lo = clf(comb, wo1_ref[...], bo1_ref[...], go_ref[...], beo_ref[...],
             wo2_ref[...], bo2_ref[...])
    oc_ref[...] = _lsm(lc[:, 0:NC])
    ot_ref[...] = _lsm(lt[:, 0:NC])
    oo_ref[...] = _lsm(lo[:, 0:NC])


def _heads(sum_c, sum_t, cnt, Wc1, bc1, gc, bec, Wc2, bc2, Wt1, bt1, gt, bet,
           Wt2, bt2, Wo1, bo1, go, beo, Wo2, bo2):
    full = lambda shp: pl.BlockSpec(shp, lambda: tuple(0 for _ in shp))
    args = [sum_c, sum_t, cnt,
            Wc1, bc1.reshape(1, HID), gc.reshape(1, HID), bec.reshape(1, HID),
            Wc2, bc2.reshape(1, NC),
            Wt1, bt1.reshape(1, HID), gt.reshape(1, HID), bet.reshape(1, HID),
            Wt2, bt2.reshape(1, NC),
            Wo1, bo1.reshape(1, HID), go.reshape(1, HID), beo.reshape(1, HID),
            Wo2, bo2.reshape(1, NC)]
    return pl.pallas_call(
        _heads_body,
        grid=(),
        in_specs=[full(a.shape) for a in args],
        out_specs=[full((G, NC))] * 3,
        out_shape=[jax.ShapeDtypeStruct((G, NC), jnp.float32)] * 3,
    )(*args)


def _att_compose(W, a_s, a_d):
    ces = (jnp.eye(HEADS, dtype=jnp.float32)[:, None, :]
           * a_s[:, :, None]).reshape(512, HEADS)
    ced = (jnp.eye(HEADS, dtype=jnp.float32)[:, None, :]
           * a_d[:, :, None]).reshape(512, HEADS)
    C = jnp.concatenate([ces, ced], axis=1)
    B = W @ C
    return jnp.pad(B, ((0, 0), (0, 112)))


def kernel(x, edge_index, batch, W0, b0, g0, be0, W1, as1, ad1, b1, g1, be1,
           W2, as2, ad2, b2, g2, be2, Wna, bna, Wea, bea, Wc, bc, Wt, bt,
           Wc1, bc1, gc, bec, Wc2, bc2, Wt1, bt1, gt, bet, Wt2, bt2,
           Wo1, bo1, go, beo, Wo2, bo2):
    src = jnp.concatenate(
        [edge_index[0].astype(jnp.int32),
         jnp.zeros((E_PAD - E,), jnp.int32)])
    dst = jnp.concatenate(
        [edge_index[1].astype(jnp.int32),
         jnp.full((E_PAD - E,), N, jnp.int32)])

    degp = _sc_deg(dst)[:, :N]
    W0p = jnp.pad(W0, ((0, 0), (0, 128 - HID)))
    h0 = _mm(x, W0p, jnp.zeros((128,), jnp.float32))
    g0tab = _gcn0_pre(h0, degp)
    accp = _sc_gcn0(g0tab, src, dst)[:, :N]
    b0p = jnp.pad(b0, (0, 128 - HID))
    r0 = _gcn0_ep(accp, degp, h0, b0p)
    st0 = _colstats(r0)
    g0p = jnp.pad(g0, (0, 128 - HID))
    be0p = jnp.pad(be0, (0, 128 - HID))
    m0, s0 = _bn_affine(st0, g0p, N)

    W1p = jnp.pad(W1, ((0, 128 - HID), (0, 0)))
    hw1, h4_1 = _mmq(r0, W1p, jnp.zeros((512,), jnp.float32), m0, s0, be0p)
    B1 = jnp.pad(_att_compose(W1, as1, ad1), ((0, 128 - HID), (0, 0)))
    t1 = _mm(r0, B1, jnp.zeros((128,), jnp.float32), m0, s0, be0p)
    p1, denp1 = _sc_att(t1, src, dst)
    acc1 = _sc_passb_gat(h4_1, src, dst, p1)[:N]
    out1 = _gat_ep(acc1, denp1[:, :N], t1, hw1, b1)
    st1 = _colstats(out1)
    m1, s1 = _bn_affine(st1, g1, N)

    hw2, h4_2 = _mmq(out1, W2, jnp.zeros((512,), jnp.float32), m1, s1, be1,
                     act="elu")
    B2 = _att_compose(W2, as2, ad2)
    t2 = _mm(out1, B2, jnp.zeros((128,), jnp.float32), m1, s1, be1, act="elu")
    p2, denp2 = _sc_att(t2, src, dst)
    acc2 = _sc_passb_gat(h4_2, src, dst, p2)[:N]
    out2 = _gat_ep(acc2, denp2[:, :N], t2, hw2, b2)
    st2 = _colstats(out2)
    m2, s2 = _bn_affine(st2, g2, N)

    hc = _mm(out2, Wc, jnp.zeros((512,), jnp.float32), m2, s2, be2, act="elu")
    ht = _mm(out2, Wt, jnp.zeros((512,), jnp.float32), m2, s2, be2, act="elu")
    B3 = jnp.pad(jnp.concatenate([Wea[:D], Wea[D:]], axis=1),
                 ((0, 0), (0, 124)))
    b3 = jnp.pad(bea, (0, 126))
    t3 = _mm(out2, B3, b3, m2, s2, be2, act="elu")
    ea, degct = _sc_ea(t3, src, dst)
    degct = degct[:, :N]
    hq8 = _ct_prescale(hc, ht, degct)
    acc_c = _sc_passb_c(hq8, src, dst, ea)[:N]
    acc_t = _sc_passb_t(hq8, src, dst, ea)[:N]
    causal, trivial = _ct_ep(acc_c, acc_t, degct, hc, ht, bc, bt)

    batch3 = batch.astype(jnp.int32).reshape(N // BM, 1, BM)
    sum_c, sum_t, cnt = _pool(batch3, causal, trivial)
    oc, ot, oco = _heads(sum_c, sum_t, cnt, Wc1, bc1, gc, bec, Wc2, bc2,
                         Wt1, bt1, gt, bet, Wt2, bt2, Wo1, bo1, go, beo,
                         Wo2, bo2)
    return (oc, ot, oco)


def _gcn0_pre_body(h_ref, d_ref, o_ref):
    deg = d_ref[0, :, 0:1] + d_ref[1, :, 0:1] + 1.0
    o_ref[...] = h_ref[...] / jnp.sqrt(deg)


def _gcn0_pre(h0, degp):
    return pl.pallas_call(
        _gcn0_pre_body,
        grid=(N // BM,),
        in_specs=[pl.BlockSpec((BM, 128), lambda i: (i, 0)),
                  pl.BlockSpec((2, BM, 128), lambda i: (0, i, 0))],
        out_specs=pl.BlockSpec((BM, 128), lambda i: (i, 0)),
        out_shape=jax.ShapeDtypeStruct((N, 128), jnp.float32),
    )(h0, degp)


# speedup vs baseline: 13.6802x; 1.1000x over previous
import functools

import jax
import jax.numpy as jnp
from jax import lax
from jax.experimental import pallas as pl
from jax.experimental.pallas import tpu as pltpu
from jax.experimental.pallas import tpu_sc as plsc

N = 10000
E = 160000
F_IN = 128
HID = 64
HEADS = 8
D = HID * HEADS
G = 64
NC = 10
EPS = 1e-5

BM = 1000
NW = 32
E_PAD = E + 256

_mesh = plsc.VectorSubcoreMesh(core_axis_name="c", subcore_axis_name="s")


def _dyng(v, idx):
    return lax.gather(
        v, idx[:, None],
        lax.GatherDimensionNumbers(offset_dims=(), collapsed_slice_dims=(0,),
                                   start_index_map=(0,)),
        (1,), mode=lax.GatherScatterMode.PROMISE_IN_BOUNDS)


def _splat(v, lane):
    return _dyng(v, jnp.full((16,), lane, jnp.int32))


def _zero_rows(z_v, rows, cols):
    def zr(i, _):
        for k in range(cols // 16):
            z_v[i, pl.ds(k * 16, 16)] = jnp.zeros((16,), jnp.float32)
        return 0
    lax.fori_loop(0, rows, zr, 0)


def _make_sc_deg(n, e):
    epw = e // NW
    npad = -(-n // 1280) * 1280
    rpt = npad // 16
    zr = 32
    nb, tail = divmod(epw, 16)

    @functools.partial(
        pl.kernel,
        out_type=jax.ShapeDtypeStruct((2, npad, 128), jnp.float32),
        mesh=_mesh,
        scratch_types=[pltpu.VMEM((epw,), jnp.int32),
                       pltpu.VMEM((16, 128), jnp.float32),
                       pltpu.VMEM((zr, 128), jnp.float32),
                       pltpu.VMEM_SHARED((npad, 128), jnp.float32),
                       pltpu.SemaphoreType.DMA])
    def k(dst_hbm, out_hbm, dst_v, one_v, z_v, acc_sh, sem):
        cid = lax.axis_index("c")
        sid = lax.axis_index("s")
        wid = cid * 16 + sid
        _zero_rows(z_v, zr, 128)
        for j in range(rpt // zr):
            pltpu.sync_copy(z_v, acc_sh.at[pl.ds(sid * rpt + j * zr, zr)])
        iota = lax.iota(jnp.int32, 16)
        ones0 = jnp.where(iota == 0, 1.0, 0.0)
        for i in range(16):
            one_v[i, pl.ds(0, 16)] = ones0
            for kk in range(1, 8):
                one_v[i, pl.ds(kk * 16, 16)] = jnp.zeros((16,), jnp.float32)
        plsc.subcore_barrier()
        pltpu.sync_copy(dst_hbm.at[pl.ds(wid * epw, epw)], dst_v)

        def blk(j, _):
            didx = dst_v[pl.ds(j * 16, 16)]
            pltpu.sync_copy(one_v, acc_sh.at[didx], add=True)
            return 0
        lax.fori_loop(0, nb, blk, 0)
        if tail:
            didx = dst_v[pl.ds(nb * 16, tail)]
            pltpu.sync_copy(one_v.at[pl.ds(0, tail)], acc_sh.at[didx], add=True)
        plsc.subcore_barrier()
        for j in range(rpt // zr):
            r0 = sid * rpt + j * zr
            pltpu.sync_copy(acc_sh.at[pl.ds(r0, zr)], out_hbm.at[cid, pl.ds(r0, zr)])
    return k


def _make_sc_gcn0(n, e):
    epw = e // NW
    npad = -(-n // 1280) * 1280
    rpt = npad // 16
    zr = 32
    nb, tail = divmod(epw, 16)

    @functools.partial(
        pl.kernel,
        out_type=jax.ShapeDtypeStruct((2, npad, 128), jnp.float32),
        mesh=_mesh,
        scratch_types=[pltpu.VMEM((epw,), jnp.int32),
                       pltpu.VMEM((epw,), jnp.int32),
                       pltpu.VMEM((16, 128), jnp.float32),
                       pltpu.VMEM((16, 128), jnp.float32),
                       pltpu.VMEM((zr, 128), jnp.float32),
                       pltpu.VMEM_SHARED((npad, 128), jnp.float32),
                       pltpu.SemaphoreType.DMA,
                       pltpu.SemaphoreType.DMA,
                       pltpu.SemaphoreType.DMA,
                       pltpu.SemaphoreType.DMA])
    def k(tab_hbm, src_hbm, dst_hbm, out_hbm, src_v, dst_v, rows_a, rows_b,
          z_v, acc_sh, semg0, semg1, sems0, sems1):
        cid = lax.axis_index("c")
        sid = lax.axis_index("s")
        wid = cid * 16 + sid
        _zero_rows(z_v, zr, 128)
        for j in range(rpt // zr):
            pltpu.sync_copy(z_v, acc_sh.at[pl.ds(sid * rpt + j * zr, zr)])
        plsc.subcore_barrier()
        pltpu.sync_copy(src_hbm.at[pl.ds(wid * epw, epw)], src_v)
        pltpu.sync_copy(dst_hbm.at[pl.ds(wid * epw, epw)], dst_v)

        def blk2(j2, _):
            j = j2 * 2
            c0 = pltpu.async_copy(
                tab_hbm.at[src_v[pl.ds(j * 16, 16)]], rows_a, semg0)
            c1 = pltpu.async_copy(
                tab_hbm.at[src_v[pl.ds(j * 16 + 16, 16)]], rows_b, semg1)
            c0.wait()
            s0 = pltpu.async_copy(
                rows_a, acc_sh.at[dst_v[pl.ds(j * 16, 16)]], sems0, add=True)
            c1.wait()
            s1 = pltpu.async_copy(
                rows_b, acc_sh.at[dst_v[pl.ds(j * 16 + 16, 16)]], sems1,
                add=True)
            s0.wait()
            s1.wait()
            return 0
        lax.fori_loop(0, nb // 2, blk2, 0)
        if nb % 2:
            j = nb - 1
            pltpu.async_copy(
                tab_hbm.at[src_v[pl.ds(j * 16, 16)]], rows_a, semg0).wait()
            pltpu.sync_copy(rows_a, acc_sh.at[dst_v[pl.ds(j * 16, 16)]],
                            add=True)
        plsc.subcore_barrier()
        for j in range(rpt // zr):
            r0 = sid * rpt + j * zr
            pltpu.sync_copy(acc_sh.at[pl.ds(r0, zr)], out_hbm.at[cid, pl.ds(r0, zr)])
    return k


def _make_sc_att(n, e):
    epw = e // NW
    npad = -(-n // 1280) * 1280
    rpt = npad // 16
    zr = 32
    nb, tail = divmod(epw, 16)

    @functools.partial(
        pl.kernel,
        out_type=(jax.ShapeDtypeStruct((e * 16,), jnp.float32),
                  jax.ShapeDtypeStruct((2, npad, 128), jnp.float32)),
        mesh=_mesh,
        scratch_types=[pltpu.VMEM((epw,), jnp.int32),
                       pltpu.VMEM((epw,), jnp.int32),
                       pltpu.VMEM((16, 128), jnp.float32),
                       pltpu.VMEM((16, 128), jnp.float32),
                       pltpu.VMEM((16, 128), jnp.float32),
                       pltpu.VMEM((16, 128), jnp.float32),
                       pltpu.VMEM((256,), jnp.float32),
                       pltpu.VMEM((256,), jnp.float32),
                       pltpu.VMEM((16, 128), jnp.float32),
                       pltpu.VMEM((16, 128), jnp.float32),
                       pltpu.VMEM((zr, 128), jnp.float32),
                       pltpu.VMEM_SHARED((npad, 128), jnp.float32),
                       pltpu.SemaphoreType.DMA,
                       pltpu.SemaphoreType.DMA,
                       pltpu.SemaphoreType.DMA,
                       pltpu.SemaphoreType.DMA,
                       pltpu.SemaphoreType.DMA,
                       pltpu.SemaphoreType.DMA])
    def k(t_hbm, src_hbm, dst_hbm, p_hbm, den_hbm,
          src_v, dst_v, rs_a, rd_a, rs_b, rd_b, pb_a, pb_b, db_a, db_b,
          z_v, acc_sh, semsa, semda, semsb, semdb, sca, scb):
        cid = lax.axis_index("c")
        sid = lax.axis_index("s")
        wid = cid * 16 + sid
        base = wid * epw
        _zero_rows(z_v, zr, 128)
        for j in range(rpt // zr):
            pltpu.sync_copy(z_v, acc_sh.at[pl.ds(sid * rpt + j * zr, zr)])
        _zero_rows(db_a, 16, 128)
        _zero_rows(db_b, 16, 128)
        plsc.subcore_barrier()
        pltpu.sync_copy(src_hbm.at[pl.ds(base, epw)], src_v)
        pltpu.sync_copy(dst_hbm.at[pl.ds(base, epw)], dst_v)
        iota = lax.iota(jnp.int32, 16)
        rot = iota % 8 + 8

        def compute(rs_v, rd_v, pb_v, db_v):
            for ee in range(16):
                es = rs_v[ee, pl.ds(0, 16)]
                ed = _dyng(rd_v[ee, pl.ds(0, 16)], rot)
                a = es + ed
                a = jnp.maximum(a, 0.2 * a)
                p = jnp.where(iota < 8, jnp.exp(a), 0.0)
                pb_v[pl.ds(ee * 16, 16)] = p
                db_v[ee, pl.ds(0, 16)] = p

        def blk2(j2, _):
            j = j2 * 2
            c0s = pltpu.async_copy(
                t_hbm.at[src_v[pl.ds(j * 16, 16)]], rs_a, semsa)
            c0d = pltpu.async_copy(
                t_hbm.at[dst_v[pl.ds(j * 16, 16)]], rd_a, semda)
            c1s = pltpu.async_copy(
                t_hbm.at[src_v[pl.ds(j * 16 + 16, 16)]], rs_b, semsb)
            c1d = pltpu.async_copy(
                t_hbm.at[dst_v[pl.ds(j * 16 + 16, 16)]], rd_b, semdb)
            c0s.wait()
            c0d.wait()
            compute(rs_a, rd_a, pb_a, db_a)
            pltpu.sync_copy(pb_a, p_hbm.at[pl.ds((base + j * 16) * 16, 256)])
            s0 = pltpu.async_copy(
                db_a, acc_sh.at[dst_v[pl.ds(j * 16, 16)]], sca, add=True)
            c1s.wait()
            c1d.wait()
            compute(rs_b, rd_b, pb_b, db_b)
            pltpu.sync_copy(pb_b,
                            p_hbm.at[pl.ds((base + j * 16 + 16) * 16, 256)])
            s1 = pltpu.async_copy(
                db_b, acc_sh.at[dst_v[pl.ds(j * 16 + 16, 16)]], scb, add=True)
            s0.wait()
            s1.wait()
            return 0
        lax.fori_loop(0, nb // 2, blk2, 0)
        if nb % 2:
            j = nb - 1
            c0s = pltpu.async_copy(
                t_hbm.at[src_v[pl.ds(j * 16, 16)]], rs_a, semsa)
            c0d = pltpu.async_copy(
                t_hbm.at[dst_v[pl.ds(j * 16, 16)]], rd_a, semda)
            c0s.wait()
            c0d.wait()
            compute(rs_a, rd_a, pb_a, db_a)
            pltpu.sync_copy(pb_a, p_hbm.at[pl.ds((base + j * 16) * 16, 256)])
            pltpu.sync_copy(db_a, acc_sh.at[dst_v[pl.ds(j * 16, 16)]],
                            add=True)
        plsc.subcore_barrier()
        for j in range(rpt // zr):
            r0 = sid * rpt + j * zr
            pltpu.sync_copy(acc_sh.at[pl.ds(r0, zr)], den_hbm.at[cid, pl.ds(r0, zr)])
    return k


def _make_sc_ea(n, e):
    epw = e // NW
    npad = -(-n // 1280) * 1280
    rpt = npad // 16
    zr = 32
    nb, tail = divmod(epw, 16)

    @functools.partial(
        pl.kernel,
        out_type=(jax.ShapeDtypeStruct((e * 16,), jnp.float32),
                  jax.ShapeDtypeStruct((2, npad, 128), jnp.float32)),
        mesh=_mesh,
        scratch_types=[pltpu.VMEM((epw,), jnp.int32),
                       pltpu.VMEM((epw,), jnp.int32),
                       pltpu.VMEM((16, 128), jnp.float32),
                       pltpu.VMEM((16, 128), jnp.float32),
                       pltpu.VMEM((16, 128), jnp.float32),
                       pltpu.VMEM((16, 128), jnp.float32),
                       pltpu.VMEM((256,), jnp.float32),
                       pltpu.VMEM((256,), jnp.float32),
                       pltpu.VMEM((16, 128), jnp.float32),
                       pltpu.VMEM((16, 128), jnp.float32),
                       pltpu.VMEM((zr, 128), jnp.float32),
                       pltpu.VMEM_SHARED((npad, 128), jnp.float32),
                       pltpu.SemaphoreType.DMA,
                       pltpu.SemaphoreType.DMA,
                       pltpu.SemaphoreType.DMA,
                       pltpu.SemaphoreType.DMA,
                       pltpu.SemaphoreType.DMA,
                       pltpu.SemaphoreType.DMA])
    def k(t_hbm, src_hbm, dst_hbm, ea_hbm, deg_hbm,
          src_v, dst_v, rs_a, rd_a, rs_b, rd_b, eb_a, eb_b, db_a, db_b,
          z_v, acc_sh, semsa, semda, semsb, semdb, sca, scb):
        cid = lax.axis_index("c")
        sid = lax.axis_index("s")
        wid = cid * 16 + sid
        base = wid * epw
        _zero_rows(z_v, zr, 128)
        for j in range(rpt // zr):
            pltpu.sync_copy(z_v, acc_sh.at[pl.ds(sid * rpt + j * zr, zr)])
        _zero_rows(db_a, 16, 128)
        _zero_rows(db_b, 16, 128)
        plsc.subcore_barrier()
        pltpu.sync_copy(src_hbm.at[pl.ds(base, epw)], src_v)
        pltpu.sync_copy(dst_hbm.at[pl.ds(base, epw)], dst_v)
        iota = lax.iota(jnp.int32, 16)
        rot2 = iota % 2 + 2

        def compute(rs_v, rd_v, eb_v, db_v):
            for ee in range(16):
                l = rs_v[ee, pl.ds(0, 16)] + _dyng(rd_v[ee, pl.ds(0, 16)], rot2)
                ldiff = _splat(l, 1) - _splat(l, 0)
                ea0 = 1.0 / (1.0 + jnp.exp(ldiff))
                ea1 = 1.0 - ea0
                eb_v[pl.ds(ee * 16, 16)] = jnp.where(iota < 8, ea0, ea1)
                db_v[ee, pl.ds(0, 16)] = jnp.where(
                    iota == 0, ea0, jnp.where(iota == 1, ea1, 0.0))

        def blk2(j2, _):
            j = j2 * 2
            c0s = pltpu.async_copy(
                t_hbm.at[src_v[pl.ds(j * 16, 16)]], rs_a, semsa)
            c0d = pltpu.async_copy(
                t_hbm.at[dst_v[pl.ds(j * 16, 16)]], rd_a, semda)
            c1s = pltpu.async_copy(
                t_hbm.at[src_v[pl.ds(j * 16 + 16, 16)]], rs_b, semsb)
            c1d = pltpu.async_copy(
                t_hbm.at[dst_v[pl.ds(j * 16 + 16, 16)]], rd_b, semdb)
            c0s.wait()
            c0d.wait()
            compute(rs_a, rd_a, eb_a, db_a)
            pltpu.sync_copy(eb_a, ea_hbm.at[pl.ds((base + j * 16) * 16, 256)])
            s0 = pltpu.async_copy(
                db_a, acc_sh.at[dst_v[pl.ds(j * 16, 16)]], sca, add=True)
            c1s.wait()
            c1d.wait()
            compute(rs_b, rd_b, eb_b, db_b)
            pltpu.sync_copy(eb_b,
                            ea_hbm.at[pl.ds((base + j * 16 + 16) * 16, 256)])
            s1 = pltpu.async_copy(
                db_b, acc_sh.at[dst_v[pl.ds(j * 16 + 16, 16)]], scb, add=True)
            s0.wait()
            s1.wait()
            return 0
        lax.fori_loop(0, nb // 2, blk2, 0)
        if nb % 2:
            j = nb - 1
            c0s = pltpu.async_copy(
                t_hbm.at[src_v[pl.ds(j * 16, 16)]], rs_a, semsa)
            c0d = pltpu.async_copy(
                t_hbm.at[dst_v[pl.ds(j * 16, 16)]], rd_a, semda)
            c0s.wait()
            c0d.wait()
            compute(rs_a, rd_a, eb_a, db_a)
            pltpu.sync_copy(eb_a, ea_hbm.at[pl.ds((base + j * 16) * 16, 256)])
            pltpu.sync_copy(db_a, acc_sh.at[dst_v[pl.ds(j * 16, 16)]],
                            add=True)
        plsc.subcore_barrier()
        for j in range(rpt // zr):
            r0 = sid * rpt + j * zr
            pltpu.sync_copy(acc_sh.at[pl.ds(r0, zr)], deg_hbm.at[cid, pl.ds(r0, zr)])
    return k


def _make_sc_passb(n, e, nq, qbase):
    ept = e // 16
    grp = 2000
    ngrp = ept // grp
    nbg = grp // 16
    npad = -(-n // 1280) * 1280
    rpt = npad // 16
    zr = 32

    nb2, blkrem = divmod(nbg, 2)

    @functools.partial(
        pl.kernel,
        out_type=jax.ShapeDtypeStruct((npad, 512), jnp.float32),
        mesh=_mesh,
        scratch_types=[pltpu.VMEM((grp,), jnp.int32),
                       pltpu.VMEM((grp,), jnp.int32),
                       pltpu.VMEM((512,), jnp.float32),
                       pltpu.VMEM((16, 128), jnp.float32),
                       pltpu.VMEM((16, 128), jnp.float32),
                       pltpu.VMEM((16, 128), jnp.float32),
                       pltpu.VMEM((16, 128), jnp.float32),
                       pltpu.VMEM((zr, 128), jnp.float32),
                       pltpu.VMEM_SHARED((npad, 128), jnp.float32),
                       pltpu.SemaphoreType.DMA,
                       pltpu.SemaphoreType.DMA,
                       pltpu.SemaphoreType.DMA,
                       pltpu.SemaphoreType.DMA])
    def k(h_hbm, src_hbm, dst_hbm, p_hbm, out_hbm,
          src_v, dst_v, p_v, rows_a, rows_b, ob_a, ob_b, z_v, acc_sh,
          semg0, semg1, sems0, sems1):
        cid = lax.axis_index("c")
        sid = lax.axis_index("s")
        ebase = sid * ept
        for kq in range(2):
            qt = cid * 2 + kq
            q = qbase + qt
            lane0 = jnp.full((16,), 2 * q, jnp.int32)
            lane1 = lane0 + 1
            _zero_rows(z_v, zr, 128)
            for j in range(rpt // zr):
                pltpu.sync_copy(z_v, acc_sh.at[pl.ds(sid * rpt + j * zr, zr)])
            plsc.subcore_barrier()

            def compute(rows_v, ob_v, poff):
                for ee in range(16):
                    pr = p_v[pl.ds(poff + ee * 16, 16)]
                    w0 = _dyng(pr, lane0)
                    w1 = _dyng(pr, lane1)
                    for v in range(4):
                        ob_v[ee, pl.ds(v * 16, 16)] = (
                            rows_v[ee, pl.ds(v * 16, 16)] * w0)
                    for v in range(4, 8):
                        ob_v[ee, pl.ds(v * 16, 16)] = (
                            rows_v[ee, pl.ds(v * 16, 16)] * w1)

            def grp_body(g, _):
                goff = ebase + g * grp
                pltpu.sync_copy(src_hbm.at[pl.ds(goff, grp)], src_v)
                pltpu.sync_copy(dst_hbm.at[pl.ds(goff, grp)], dst_v)

                def blk2(j2, _):
                    j = j2 * 2
                    c0 = pltpu.async_copy(
                        h_hbm.at[q].at[src_v[pl.ds(j * 16, 16)]], rows_a,
                        semg0)
                    c1 = pltpu.async_copy(
                        h_hbm.at[q].at[src_v[pl.ds(j * 16 + 16, 16)]], rows_b,
                        semg1)
                    pltpu.sync_copy(p_hbm.at[pl.ds((goff + j * 16) * 16, 512)],
                                    p_v)
                    c0.wait()
                    compute(rows_a, ob_a, 0)
                    s0 = pltpu.async_copy(
                        ob_a, acc_sh.at[dst_v[pl.ds(j * 16, 16)]], sems0,
                        add=True)
                    c1.wait()
                    compute(rows_b, ob_b, 256)
                    s1 = pltpu.async_copy(
                        ob_b, acc_sh.at[dst_v[pl.ds(j * 16 + 16, 16)]], sems1,
                        add=True)
                    s0.wait()
                    s1.wait()
                    return 0
                lax.fori_loop(0, nb2, blk2, 0)
                if blkrem:
                    j = nb2 * 2
                    cg = pltpu.async_copy(
                        h_hbm.at[q].at[src_v[pl.ds(j * 16, 16)]], rows_a,
                        semg0)
                    pltpu.sync_copy(p_hbm.at[pl.ds((goff + j * 16) * 16, 256)],
                                    p_v.at[pl.ds(0, 256)])
                    cg.wait()
                    compute(rows_a, ob_a, 0)
                    pltpu.sync_copy(ob_a, acc_sh.at[dst_v[pl.ds(j * 16, 16)]],
                                    add=True)
                return 0
            lax.fori_loop(0, ngrp, grp_body, 0)
            plsc.subcore_barrier()
            for j in range(rpt // zr):
                r0 = sid * rpt + j * zr
                pltpu.sync_copy(acc_sh.at[pl.ds(r0, zr)],
                                out_hbm.at[pl.ds(r0, zr), pl.ds(qt * 128, 128)])
            plsc.subcore_barrier()
    return k


_sc_deg = _make_sc_deg(N, E_PAD)
_sc_gcn0 = _make_sc_gcn0(N, E_PAD)
_sc_att = _make_sc_att(N, E_PAD)
_sc_ea = _make_sc_ea(N, E_PAD)
_sc_passb_gat = _make_sc_passb(N, E, 4, 0)
_sc_passb_c = _make_sc_passb(N, E, 8, 0)
_sc_passb_t = _make_sc_passb(N, E, 8, 4)


def _mm_body(x_ref, w_ref, b_ref, m_ref, s_ref, e_ref, o_ref, *, act):
    x = (x_ref[...] - m_ref[...]) * s_ref[...] + e_ref[...]
    if act == "elu":
        x = jnp.where(x > 0, x, jnp.exp(jnp.minimum(x, 0.0)) - 1.0)
    o_ref[...] = jnp.dot(x, w_ref[...], preferred_element_type=jnp.float32) \
        + b_ref[...]


def _mm(x, w, b, m=None, s=None, be=None, act="none"):
    n, kdim = x.shape
    f = w.shape[1]
    if m is None:
        m = jnp.zeros((kdim,), jnp.float32)
    if s is None:
        s = jnp.ones((kdim,), jnp.float32)
    if be is None:
        be = jnp.zeros((kdim,), jnp.float32)
    grid = (n // BM, f // 128)
    return pl.pallas_call(
        functools.partial(_mm_body, act=act),
        grid=grid,
        in_specs=[pl.BlockSpec((BM, kdim), lambda i, j: (i, 0)),
                  pl.BlockSpec((kdim, 128), lambda i, j: (0, j)),
                  pl.BlockSpec((1, 128), lambda i, j: (0, j)),
                  pl.BlockSpec((1, kdim), lambda i, j: (0, 0)),
                  pl.BlockSpec((1, kdim), lambda i, j: (0, 0)),
                  pl.BlockSpec((1, kdim), lambda i, j: (0, 0))],
        out_specs=pl.BlockSpec((BM, 128), lambda i, j: (i, j)),
        out_shape=jax.ShapeDtypeStruct((n, f), jnp.float32),
    )(x, w, b.reshape(1, f), m.reshape(1, kdim), s.reshape(1, kdim),
      be.reshape(1, kdim))


def _mmq_body(x_ref, w_ref, b_ref, m_ref, s_ref, e_ref, o_ref, q_ref, *, act):
    x = (x_ref[...] - m_ref[...]) * s_ref[...] + e_ref[...]
    if act == "elu":
        x = jnp.where(x > 0, x, jnp.exp(jnp.minimum(x, 0.0)) - 1.0)
    r = jnp.dot(x, w_ref[...], preferred_element_type=jnp.float32) + b_ref[...]
    o_ref[...] = r
    q_ref[...] = r[None]


def _mmq(x, w, b, m=None, s=None, be=None, act="none"):
    n, kdim = x.shape
    f = w.shape[1]
    if m is None:
        m = jnp.zeros((kdim,), jnp.float32)
    if s is None:
        s = jnp.ones((kdim,), jnp.float32)
    if be is None:
        be = jnp.zeros((kdim,), jnp.float32)
    grid = (n // BM, f // 128)
    return pl.pallas_call(
        functools.partial(_mmq_body, act=act),
        grid=grid,
        in_specs=[pl.BlockSpec((BM, kdim), lambda i, j: (i, 0)),
                  pl.BlockSpec((kdim, 128), lambda i, j: (0, j)),
                  pl.BlockSpec((1, 128), lambda i, j: (0, j)),
                  pl.BlockSpec((1, kdim), lambda i, j: (0, 0)),
                  pl.BlockSpec((1, kdim), lambda i, j: (0, 0)),
                  pl.BlockSpec((1, kdim), lambda i, j: (0, 0))],
        out_specs=[pl.BlockSpec((BM, 128), lambda i, j: (i, j)),
                   pl.BlockSpec((1, BM, 128), lambda i, j: (j, i, 0))],
        out_shape=[jax.ShapeDtypeStruct((n, f), jnp.float32),
                   jax.ShapeDtypeStruct((f // 128, n, 128), jnp.float32)],
    )(x, w, b.reshape(1, f), m.reshape(1, kdim), s.reshape(1, kdim),
      be.reshape(1, kdim))


def _stats_body(x_ref, o_ref):
    i = pl.program_id(0)
    x = x_ref[...]
    s = jnp.sum(x, axis=0, keepdims=True)
    s2 = jnp.sum(x * x, axis=0, keepdims=True)
    blk = jnp.concatenate([s, s2, jnp.zeros((6, x.shape[1]), jnp.float32)], 0)

    @pl.when(i == 0)
    def _():
        o_ref[...] = blk

    @pl.when(i > 0)
    def _():
        o_ref[...] = o_ref[...] + blk


def _colstats(x):
    n, f = x.shape
    return pl.pallas_call(
        _stats_body,
        grid=(n // BM,),
        in_specs=[pl.BlockSpec((BM, f), lambda i: (i, 0))],
        out_specs=pl.BlockSpec((8, f), lambda i: (0, 0)),
        out_shape=jax.ShapeDtypeStruct((8, f), jnp.float32),
    )(x)


def _bn_affine(stats, g, n):
    mean = stats[0] / n
    var = stats[1] / n - mean * mean
    return mean, g / jnp.sqrt(var + EPS)


def _gcn0_ep_body(a_ref, d_ref, h_ref, b_ref, o_ref):
    deg = d_ref[0, :, 0:1] + d_ref[1, :, 0:1] + 1.0
    dis = 1.0 / jnp.sqrt(deg)
    acc = a_ref[0] + a_ref[1]
    o_ref[...] = jnp.maximum(dis * acc + dis * dis * h_ref[...] + b_ref[...],
                             0.0)


def _gcn0_ep(accp, degp, h0, b0p):
    return pl.pallas_call(
        _gcn0_ep_body,
        grid=(N // BM,),
        in_specs=[pl.BlockSpec((2, BM, 128), lambda i: (0, i, 0)),
                  pl.BlockSpec((2, BM, 128), lambda i: (0, i, 0)),
                  pl.BlockSpec((BM, 128), lambda i: (i, 0)),
                  pl.BlockSpec((1, 128), lambda i: (0, 0))],
        out_specs=pl.BlockSpec((BM, 128), lambda i: (i, 0)),
        out_shape=jax.ShapeDtypeStruct((N, 128), jnp.float32),
    )(accp, degp, h0, b0p.reshape(1, 128))


def _expand_heads(v8, bm):
    return jnp.concatenate(
        [jnp.broadcast_to(v8[:, k:k + 1], (bm, HID)) for k in range(HEADS)], 1)


def _gat_ep_body(acc_ref, dn_ref, t_ref, hw_ref, b_ref, o_ref):
    t = t_ref[...]
    es = t[:, 0:8]
    ed = t[:, 8:16]
    a = es + ed
    p_self = jnp.exp(jnp.maximum(a, 0.2 * a))
    den8 = dn_ref[0, :, 0:8] + dn_ref[1, :, 0:8] + p_self
    den = _expand_heads(den8, acc_ref.shape[0])
    ps = _expand_heads(p_self, acc_ref.shape[0])
    o_ref[...] = (acc_ref[...] + ps * hw_ref[...]) / den + b_ref[...]


def _gat_ep(acc, denp, t, hw, b):
    return pl.pallas_call(
        _gat_ep_body,
        grid=(N // BM,),
        in_specs=[pl.BlockSpec((BM, 512), lambda i: (i, 0)),
                  pl.BlockSpec((2, BM, 128), lambda i: (0, i, 0)),
                  pl.BlockSpec((BM, 128), lambda i: (i, 0)),
                  pl.BlockSpec((BM, 512), lambda i: (i, 0)),
                  pl.BlockSpec((1, 512), lambda i: (0, 0))],
        out_specs=pl.BlockSpec((BM, 512), lambda i: (i, 0)),
        out_shape=jax.ShapeDtypeStruct((N, 512), jnp.float32),
    )(acc, denp, t, hw, b.reshape(1, 512))


def _ct_pre_body(hc_ref, ht_ref, d_ref, o_ref):
    j = pl.program_id(1)
    lane = d_ref[0, :, 0:2] + d_ref[1, :, 0:2] + 1.0
    dis_c = 1.0 / jnp.sqrt(lane[:, 0:1])
    dis_t = 1.0 / jnp.sqrt(lane[:, 1:2])
    dis = jnp.where(j < 4, dis_c, dis_t)
    h = jnp.where(j < 4, hc_ref[...], ht_ref[...])
    o_ref[...] = (dis * h)[None]


def _ct_prescale(hc, ht, degp):
    return pl.pallas_call(
        _ct_pre_body,
        grid=(N // BM, 8),
        in_specs=[pl.BlockSpec((BM, 128), lambda i, j: (i, j % 4)),
                  pl.BlockSpec((BM, 128), lambda i, j: (i, j % 4)),
                  pl.BlockSpec((2, BM, 128), lambda i, j: (0, i, 0))],
        out_specs=pl.BlockSpec((1, BM, 128), lambda i, j: (j, i, 0)),
        out_shape=jax.ShapeDtypeStruct((8, N, 128), jnp.float32),
    )(hc, ht, degp)


def _ct_ep_body(ac_ref, at_ref, d_ref, hc_ref, ht_ref, bc_ref,
                bt_ref, oc_ref, ot_ref):
    lane = d_ref[0, :, 0:2] + d_ref[1, :, 0:2] + 1.0
    dis_c = 1.0 / jnp.sqrt(lane[:, 0:1])
    dis_t = 1.0 / jnp.sqrt(lane[:, 1:2])
    oc_ref[...] = dis_c * ac_ref[...] + dis_c * dis_c * hc_ref[...] + bc_ref[...]
    ot_ref[...] = dis_t * at_ref[...] + dis_t * dis_t * ht_ref[...] + bt_ref[...]


def _ct_ep(acc_c, acc_t, degp, hc, ht, bc, bt):
    return pl.pallas_call(
        _ct_ep_body,
        grid=(N // BM,),
        in_specs=[pl.BlockSpec((BM, 512), lambda i: (i, 0)),
                  pl.BlockSpec((BM, 512), lambda i: (i, 0)),
                  pl.BlockSpec((2, BM, 128), lambda i: (0, i, 0)),
                  pl.BlockSpec((BM, 512), lambda i: (i, 0)),
                  pl.BlockSpec((BM, 512), lambda i: (i, 0)),
                  pl.BlockSpec((1, 512), lambda i: (0, 0)),
                  pl.BlockSpec((1, 512), lambda i: (0, 0))],
        out_specs=[pl.BlockSpec((BM, 512), lambda i: (i, 0)),
                   pl.BlockSpec((BM, 512), lambda i: (i, 0))],
        out_shape=[jax.ShapeDtypeStruct((N, 512), jnp.float32),
                   jax.ShapeDtypeStruct((N, 512), jnp.float32)],
    )(acc_c, acc_t, degp, hc, ht,
      bc.reshape(1, 512), bt.reshape(1, 512))


def _pool_body(b_ref, c_ref, t_ref, oc_ref, ot_ref, on_ref):
    i = pl.program_id(0)
    batch = b_ref[0, 0, :]
    gi = lax.broadcasted_iota(jnp.int32, (G, BM), 0)
    oh = (gi == batch[None, :]).astype(jnp.float32)
    pc = jnp.dot(oh, c_ref[...], preferred_element_type=jnp.float32)
    pt = jnp.dot(oh, t_ref[...], preferred_element_type=jnp.float32)
    cnt = jnp.concatenate([jnp.sum(oh, axis=1, keepdims=True),
                           jnp.zeros((G, 127), jnp.float32)], 1)

    @pl.when(i == 0)
    def _():
        oc_ref[...] = pc
        ot_ref[...] = pt
        on_ref[...] = cnt

    @pl.when(i > 0)
    def _():
        oc_ref[...] = oc_ref[...] + pc
        ot_ref[...] = ot_ref[...] + pt
        on_ref[...] = on_ref[...] + cnt


def _pool(batch3, causal, trivial):
    return pl.pallas_call(
        _pool_body,
        grid=(N // BM,),
        in_specs=[pl.BlockSpec((1, 1, BM), lambda i: (i, 0, 0)),
                  pl.BlockSpec((BM, 512), lambda i: (i, 0)),
                  pl.BlockSpec((BM, 512), lambda i: (i, 0))],
        out_specs=[pl.BlockSpec((G, 512), lambda i: (0, 0)),
                   pl.BlockSpec((G, 512), lambda i: (0, 0)),
                   pl.BlockSpec((G, 128), lambda i: (0, 0))],
        out_shape=[jax.ShapeDtypeStruct((G, 512), jnp.float32),
                   jax.ShapeDtypeStruct((G, 512), jnp.float32),
                   jax.ShapeDtypeStruct((G, 128), jnp.float32)],
    )(batch3, causal, trivial)


def _bn64(x, g, be):
    m = jnp.mean(x, axis=0, keepdims=True)
    v = jnp.mean(x * x, axis=0, keepdims=True) - m * m
    return (x - m) / jnp.sqrt(v + EPS) * g + be


def _lsm(logits):
    lm = jnp.max(logits, axis=-1, keepdims=True)
    return logits - lm - jnp.log(jnp.sum(jnp.exp(logits - lm), axis=-1,
                                         keepdims=True))


def _heads_body(sc_ref, st_ref, cnt_ref, wc1_ref, bc1_ref, gc_ref, bec_ref,
                wc2_ref, bc2_ref, wt1_ref, bt1_ref, gt_ref, bet_ref,
                wt2_ref, bt2_ref, wo1_ref, bo1_ref, go_ref, beo_ref,
                wo2_ref, bo2_ref, oc_ref, ot_ref, oo_ref):
    cnt = jnp.maximum(cnt_ref[...][:, 0:1], 1.0)
    pc = sc_ref[...] / cnt
    pt = st_ref[...] / cnt

    def clf(x, w1, b1, g, be, w2, b2):
        h = jnp.maximum(_bn64(
            jnp.dot(x, w1, preferred_element_type=jnp.float32) + b1, g, be), 0.0)
        return jnp.dot(h, w2, preferred_element_type=jnp.float32) + b2

    lc = clf(pc, wc1_ref[...], bc1_ref[...], gc_ref[...], bec_ref[...],
             wc2_ref[...], bc2_ref[...])
    lt = clf(pt, wt1_ref[...], bt1_ref[...], gt_ref[...], bet_ref[...],
             wt2_ref[...], bt2_ref[...])
    comb = jnp.concatenate([pc, pt], axis=1)
    lo = clf(comb, wo1_ref[...], bo1_ref[...], go_ref[...], beo_ref[...],
             wo2_ref[...], bo2_ref[...])
    oc_ref[...] = _lsm(lc[:, 0:NC])
    ot_ref[...] = _lsm(lt[:, 0:NC])
    oo_ref[...] = _lsm(lo[:, 0:NC])


def _heads(sum_c, sum_t, cnt, Wc1, bc1, gc, bec, Wc2, bc2, Wt1, bt1, gt, bet,
           Wt2, bt2, Wo1, bo1, go, beo, Wo2, bo2):
    full = lambda shp: pl.BlockSpec(shp, lambda: tuple(0 for _ in shp))
    args = [sum_c, sum_t, cnt,
            Wc1, bc1.reshape(1, HID), gc.reshape(1, HID), bec.reshape(1, HID),
            Wc2, bc2.reshape(1, NC),
            Wt1, bt1.reshape(1, HID), gt.reshape(1, HID), bet.reshape(1, HID),
            Wt2, bt2.reshape(1, NC),
            Wo1, bo1.reshape(1, HID), go.reshape(1, HID), beo.reshape(1, HID),
            Wo2, bo2.reshape(1, NC)]
    return pl.pallas_call(
        _heads_body,
        grid=(),
        in_specs=[full(a.shape) for a in args],
        out_specs=[full((G, NC))] * 3,
        out_shape=[jax.ShapeDtypeStruct((G, NC), jnp.float32)] * 3,
    )(*args)


def _att_compose(W, a_s, a_d):
    ces = (jnp.eye(HEADS, dtype=jnp.float32)[:, None, :]
           * a_s[:, :, None]).reshape(512, HEADS)
    ced = (jnp.eye(HEADS, dtype=jnp.float32)[:, None, :]
           * a_d[:, :, None]).reshape(512, HEADS)
    C = jnp.concatenate([ces, ced], axis=1)
    B = W @ C
    return jnp.pad(B, ((0, 0), (0, 112)))


def kernel(x, edge_index, batch, W0, b0, g0, be0, W1, as1, ad1, b1, g1, be1,
           W2, as2, ad2, b2, g2, be2, Wna, bna, Wea, bea, Wc, bc, Wt, bt,
           Wc1, bc1, gc, bec, Wc2, bc2, Wt1, bt1, gt, bet, Wt2, bt2,
           Wo1, bo1, go, beo, Wo2, bo2):
    src = jnp.concatenate(
        [edge_index[0].astype(jnp.int32),
         jnp.zeros((E_PAD - E,), jnp.int32)])
    dst = jnp.concatenate(
        [edge_index[1].astype(jnp.int32),
         jnp.full((E_PAD - E,), N, jnp.int32)])

    degp = _sc_deg(dst)[:, :N]
    W0p = jnp.pad(W0, ((0, 0), (0, 128 - HID)))
    h0 = _mm(x, W0p, jnp.zeros((128,), jnp.float32))
    g0tab = _gcn0_pre(h0, degp)
    accp = _sc_gcn0(g0tab, src, dst)[:, :N]
    b0p = jnp.pad(b0, (0, 128 - HID))
    r0 = _gcn0_ep(accp, degp, h0, b0p)
    st0 = _colstats(r0)
    g0p = jnp.pad(g0, (0, 128 - HID))
    be0p = jnp.pad(be0, (0, 128 - HID))
    m0, s0 = _bn_affine(st0, g0p, N)

    W1p = jnp.pad(W1, ((0, 128 - HID), (0, 0)))
    hw1, h4_1 = _mmq(r0, W1p, jnp.zeros((512,), jnp.float32), m0, s0, be0p)
    B1 = jnp.pad(_att_compose(W1, as1, ad1), ((0, 128 - HID), (0, 0)))
    t1 = _mm(r0, B1, jnp.zeros((128,), jnp.float32), m0, s0, be0p)
    p1, denp1 = _sc_att(t1, src, dst)
    acc1 = _sc_passb_gat(h4_1, src, dst, p1)[:N]
    out1 = _gat_ep(acc1, denp1[:, :N], t1, hw1, b1)
    st1 = _colstats(out1)
    m1, s1 = _bn_affine(st1, g1, N)

    hw2, h4_2 = _mmq(out1, W2, jnp.zeros((512,), jnp.float32), m1, s1, be1,
                     act="elu")
    B2 = _att_compose(W2, as2, ad2)
    t2 = _mm(out1, B2, jnp.zeros((128,), jnp.float32), m1, s1, be1, act="elu")
    p2, denp2 = _sc_att(t2, src, dst)
    acc2 = _sc_passb_gat(h4_2, src, dst, p2)[:N]
    out2 = _gat_ep(acc2, denp2[:, :N], t2, hw2, b2)
    st2 = _colstats(out2)
    m2, s2 = _bn_affine(st2, g2, N)

    hc = _mm(out2, Wc, jnp.zeros((512,), jnp.float32), m2, s2, be2, act="elu")
    ht = _mm(out2, Wt, jnp.zeros((512,), jnp.float32), m2, s2, be2, act="elu")
    B3 = jnp.pad(jnp.concatenate([Wea[:D], Wea[D:]], axis=1),
                 ((0, 0), (0, 124)))
    b3 = jnp.pad(bea, (0, 126))
    t3 = _mm(out2, B3, b3, m2, s2, be2, act="elu")
    ea, degct = _sc_ea(t3, src, dst)
    degct = degct[:, :N]
    hq8 = _ct_prescale(hc, ht, degct)
    acc_c = _sc_passb_c(hq8, src, dst, ea)[:N]
    acc_t = _sc_passb_t(hq8, src, dst, ea)[:N]
    causal, trivial = _ct_ep(acc_c, acc_t, degct, hc, ht, bc, bt)

    batch3 = batch.astype(jnp.int32).reshape(N // BM, 1, BM)
    sum_c, sum_t, cnt = _pool(batch3, causal, trivial)
    oc, ot, oco = _heads(sum_c, sum_t, cnt, Wc1, bc1, gc, bec, Wc2, bc2,
                         Wt1, bt1, gt, bet, Wt2, bt2, Wo1, bo1, go, beo,
                         Wo2, bo2)
    return (oc, ot, oco)


def _gcn0_pre_body(h_ref, d_ref, o_ref):
    deg = d_ref[0, :, 0:1] + d_ref[1, :, 0:1] + 1.0
    o_ref[...] = h_ref[...] / jnp.sqrt(deg)


def _gcn0_pre(h0, degp):
    return pl.pallas_call(
        _gcn0_pre_body,
        grid=(N // BM,),
        in_specs=[pl.BlockSpec((BM, 128), lambda i: (i, 0)),
                  pl.BlockSpec((2, BM, 128), lambda i: (0, i, 0))],
        out_specs=pl.BlockSpec((BM, 128), lambda i: (i, 0)),
        out_shape=jax.ShapeDtypeStruct((N, 128), jnp.float32),
    )(h0, degp)
